# baseline (speedup 1.0000x reference)
import functools
import jax
import jax.numpy as jnp
from jax import lax
from jax.experimental import pallas as pl
from jax.experimental.pallas import tpu as pltpu

_N = 10000
_E = 320000
_EPAD = 327680
_H = 128
_NG = 64
_EPS = 1e-8
_BIG = 1e30


def _trig_gf_body(vd_ref, w1_ref, b1_ref, w2_ref, b2_ref, o1a, o1b, o2a, o2b):
    vd = vd_ref[...]
    v1x, v1y, v1z = vd[:, 0:1], vd[:, 1:2], vd[:, 2:3]
    v2x, v2y, v2z = vd[:, 3:4], vd[:, 4:5], vd[:, 5:6]
    v3x, v3y, v3z = vd[:, 6:7], vd[:, 7:8], vd[:, 8:9]
    d2 = vd[:, 9:10]
    d = jnp.sqrt(d2)
    ct = jnp.clip(v2z / (d + _EPS), -1.0 + _EPS, 1.0 - _EPS)
    theta = jnp.arctan2(jnp.sqrt(jnp.maximum(1.0 - ct * ct, 0.0)), ct)
    phi = jnp.arctan2(v2y, v2x)
    n1x = v1y * v2z - v1z * v2y
    n1y = v1z * v2x - v1x * v2z
    n1z = v1x * v2y - v1y * v2x
    n2x = v2y * v3z - v2z * v3y
    n2y = v2z * v3x - v2x * v3z
    n2z = v2x * v3y - v2y * v3x
    dotn = n1x * n2x + n1y * n2y + n1z * n2z
    nn1 = jnp.sqrt(n1x * n1x + n1y * n1y + n1z * n1z) + _EPS
    nn2 = jnp.sqrt(n2x * n2x + n2y * n2y + n2z * n2z) + _EPS
    ctau = jnp.clip(dotn / (nn1 * nn2), -1.0 + _EPS, 1.0 - _EPS)
    tau = jnp.arctan2(jnp.sqrt(jnp.maximum(1.0 - ctau * ctau, 0.0)), ctau)

    def gf(w_ref, b_ref):
        w = w_ref[...]
        return jnp.maximum(d * w[0:1, :] + theta * w[1:2, :] + phi * w[2:3, :]
                           + tau * w[3:4, :] + b_ref[...], 0.0)

    g1 = gf(w1_ref, b1_ref)
    g2 = gf(w2_ref, b2_ref)
    o1a[...] = g1[:, :64]
    o1b[...] = g1[:, 64:]
    o2a[...] = g2[:, :64]
    o2b[...] = g2[:, 64:]


def _trig_gf(vd, w1, b1, w2, b2):
    be = 4096
    grid = (_EPAD // be,)
    half = jax.ShapeDtypeStruct((_EPAD, 64), jnp.float32)
    return pl.pallas_call(
        _trig_gf_body,
        grid=grid,
        in_specs=[
            pl.BlockSpec((be, 16), lambda i: (i, 0)),
            pl.BlockSpec((4, _H), lambda i: (0, 0)),
            pl.BlockSpec((1, _H), lambda i: (0, 0)),
            pl.BlockSpec((4, _H), lambda i: (0, 0)),
            pl.BlockSpec((1, _H), lambda i: (0, 0)),
        ],
        out_specs=[pl.BlockSpec((be, 64), lambda i: (i, 0))] * 4,
        out_shape=[half, half, half, half],
    )(vd, w1, b1, w2, b2)


def _nodelin_body(h_ref, w_ref, b_ref, oa, ob):
    hh = jnp.dot(h_ref[...], w_ref[...],
                 preferred_element_type=jnp.float32) + b_ref[...]
    oa[...] = hh[:, :64]
    ob[...] = hh[:, 64:]


def _nodelin(h, w, b):
    bn = 2000
    half = jax.ShapeDtypeStruct((_N, 64), jnp.float32)
    return pl.pallas_call(
        _nodelin_body,
        grid=(_N // bn,),
        in_specs=[
            pl.BlockSpec((bn, _H), lambda i: (i, 0)),
            pl.BlockSpec((_H, _H), lambda i: (0, 0)),
            pl.BlockSpec((1, _H), lambda i: (0, 0)),
        ],
        out_specs=[pl.BlockSpec((bn, 64), lambda i: (i, 0))] * 2,
        out_shape=[half, half],
    )(h, w, b)


def _combine_body(s1a, s1b, aa, ab, deg_ref, g2wa, g2wb, g2b, lw, lb, oa, ob):
    acont = (jnp.dot(aa[...], g2wa[...], preferred_element_type=jnp.float32)
             + jnp.dot(ab[...], g2wb[...], preferred_element_type=jnp.float32))
    s1 = jnp.concatenate([s1a[...], s1b[...]], axis=1)
    h = jnp.maximum(s1 + acont + deg_ref[...] * g2b[...], 0.0)
    hh = jnp.dot(h, lw[...], preferred_element_type=jnp.float32) + lb[...]
    oa[...] = hh[:, :64]
    ob[...] = hh[:, 64:]


def _combine_lin(s1a, s1b, aa, ab, deg, g2w, g2b, lw, lb):
    bn = 2000
    half = jax.ShapeDtypeStruct((_N, 64), jnp.float32)
    return pl.pallas_call(
        _combine_body,
        grid=(_N // bn,),
        in_specs=[
            pl.BlockSpec((bn, 64), lambda i: (i, 0)),
            pl.BlockSpec((bn, 64), lambda i: (i, 0)),
            pl.BlockSpec((bn, 64), lambda i: (i, 0)),
            pl.BlockSpec((bn, 64), lambda i: (i, 0)),
            pl.BlockSpec((bn, 1), lambda i: (i, 0)),
            pl.BlockSpec((64, _H), lambda i: (0, 0)),
            pl.BlockSpec((64, _H), lambda i: (0, 0)),
            pl.BlockSpec((1, _H), lambda i: (0, 0)),
            pl.BlockSpec((_H, _H), lambda i: (0, 0)),
            pl.BlockSpec((1, _H), lambda i: (0, 0)),
        ],
        out_specs=[pl.BlockSpec((bn, 64), lambda i: (i, 0))] * 2,
        out_shape=[half, half],
    )(s1a, s1b, aa, ab, deg, g2w[:64], g2w[64:], g2b, lw, lb)


def _final_body(s1a, s1b, aa, ab, deg_ref, g2wa, g2wb, g2b, saw, sab,
                oh_ref, l1w, l1b, l2w, l2b, sums_ref, cnt_ref, out_ref):
    i = pl.program_id(0)
    nsteps = pl.num_programs(0)
    acont = (jnp.dot(aa[...], g2wa[...], preferred_element_type=jnp.float32)
             + jnp.dot(ab[...], g2wb[...], preferred_element_type=jnp.float32))
    s1 = jnp.concatenate([s1a[...], s1b[...]], axis=1)
    h2 = jnp.maximum(s1 + acont + deg_ref[...] * g2b[...], 0.0)
    h3 = jnp.maximum(jnp.dot(h2, saw[...],
                             preferred_element_type=jnp.float32) + sab[...], 0.0)
    oh = oh_ref[...]
    psum = jnp.dot(oh, h3, preferred_element_type=jnp.float32)
    ones = jnp.ones((oh.shape[1], _H), jnp.float32)
    pcnt = jnp.dot(oh, ones, preferred_element_type=jnp.float32)

    @pl.when(i == 0)
    def _():
        sums_ref[...] = psum
        cnt_ref[...] = pcnt

    @pl.when(i > 0)
    def _():
        sums_ref[...] += psum
        cnt_ref[...] += pcnt

    @pl.when(i == nsteps - 1)
    def _():
        cnt = cnt_ref[...]
        sums = sums_ref[...]
        pooled = jnp.where(cnt > 0, sums / jnp.maximum(cnt, 1.0), 0.0)
        hf = jnp.maximum(jnp.dot(pooled, l1w[...],
                                 preferred_element_type=jnp.float32) + l1b[...], 0.0)
        out_ref[...] = jnp.dot(hf, l2w[...],
                               preferred_element_type=jnp.float32) + l2b[...]


def _final_head(s1a, s1b, aa, ab, deg, g2w, g2b, saw, sab, onehot,
                l1w, l1b, l2wp, l2bp):
    bn = 2048
    npad = 10240
    pad = lambda t: jnp.pad(t, ((0, npad - _N), (0, 0)))
    s1a, s1b, aa, ab, deg = map(pad, (s1a, s1b, aa, ab, deg))
    onehot = jnp.pad(onehot, ((0, 0), (0, npad - _N)))
    outs = [jax.ShapeDtypeStruct((_NG, _H), jnp.float32),
            jax.ShapeDtypeStruct((_NG, _H), jnp.float32),
            jax.ShapeDtypeStruct((_NG, _H), jnp.float32)]
    sums, cnt, out = pl.pallas_call(
        _final_body,
        grid=(npad // bn,),
        in_specs=[
            pl.BlockSpec((bn, 64), lambda i: (i, 0)),
            pl.BlockSpec((bn, 64), lambda i: (i, 0)),
            pl.BlockSpec((bn, 64), lambda i: (i, 0)),
            pl.BlockSpec((bn, 64), lambda i: (i, 0)),
            pl.BlockSpec((bn, 1), lambda i: (i, 0)),
            pl.BlockSpec((64, _H), lambda i: (0, 0)),
            pl.BlockSpec((64, _H), lambda i: (0, 0)),
            pl.BlockSpec((1, _H), lambda i: (0, 0)),
            pl.BlockSpec((_H, _H), lambda i: (0, 0)),
            pl.BlockSpec((1, _H), lambda i: (0, 0)),
            pl.BlockSpec((_NG, bn), lambda i: (0, i)),
            pl.BlockSpec((_H, 64), lambda i: (0, 0)),
            pl.BlockSpec((1, 64), lambda i: (0, 0)),
            pl.BlockSpec((64, _H), lambda i: (0, 0)),
            pl.BlockSpec((1, _H), lambda i: (0, 0)),
        ],
        out_specs=[pl.BlockSpec((_NG, _H), lambda i: (0, 0))] * 3,
        out_shape=outs,
    )(s1a, s1b, aa, ab, deg, g2w[:64], g2w[64:], g2b, saw, sab, onehot,
      l1w, l1b, l2wp, l2bp)
    return out[:, :1]


def _edge_structure(pos, src, dst):
    rel = pos[dst] - pos[src]
    d2e = jnp.sum(rel * rel, axis=-1)
    min1 = jax.ops.segment_min(d2e, src, num_segments=_N)
    nn1 = jax.ops.segment_min(jnp.where(d2e <= min1[src], dst, _N), src,
                              num_segments=_N)
    m2 = dst != nn1[src]
    d2x = jnp.where(m2, d2e, _BIG)
    min2 = jax.ops.segment_min(d2x, src, num_segments=_N)
    nn2 = jax.ops.segment_min(jnp.where(d2x <= min2[src], dst, _N), src,
                              num_segments=_N)
    has1 = min1 < _BIG
    has2 = min2 < _BIG
    nn1c = jnp.minimum(nn1, _N - 1)
    nn2c = jnp.minimum(nn2, _N - 1)
    f1s, f2s, h1s, h2s = nn1c[src], nn2c[src], has1[src], has2[src]
    fi = jnp.where(~h1s, src, jnp.where(f1s != dst, f1s,
                                        jnp.where(h2s, f2s, src)))
    f1d, f2d, h1d, h2d = nn1c[dst], nn2c[dst], has1[dst], has2[dst]
    fj = jnp.where(~h1d, dst, jnp.where(f1d != src, f1d,
                                        jnp.where(h2d, f2d, dst)))
    v1 = pos[src] - pos[fi]
    v3 = pos[dst] - pos[fj]
    vd = jnp.concatenate([v1, rel, v3, d2e[:, None]], axis=1)
    vd = jnp.pad(vd, ((0, _EPAD - _E), (0, 6)))
    return vd


def _gather_scatter(tbl_a, tbl_b, src, dst):
    sa = jax.ops.segment_sum(tbl_a[src], dst, num_segments=_N)
    sb = jax.ops.segment_sum(tbl_b[src], dst, num_segments=_N)
    return sa, sb


def _scatter_rows(rows_a, rows_b, dst):
    aa = jax.ops.segment_sum(rows_a[:_E], dst, num_segments=_N)
    ab = jax.ops.segment_sum(rows_b[:_E], dst, num_segments=_N)
    return aa, ab


def kernel(x, edge_index, batch, pos, c1_lin_w, c1_lin_b, c1_g1_w, c1_g1_b,
           c1_g2_w, c1_g2_b, c2_lin_w, c2_lin_b, c2_g1_w, c2_g1_b, c2_g2_w,
           c2_g2_b, sa_w, sa_b, l1_w, l1_b, l2_w, l2_b):
    src = edge_index[0].astype(jnp.int32)
    dst = edge_index[1].astype(jnp.int32)

    vd = _edge_structure(pos, src, dst)
    gf1a, gf1b, gf2a, gf2b = _trig_gf(
        vd, c1_g1_w, c1_g1_b.reshape(1, _H), c2_g1_w, c2_g1_b.reshape(1, _H))

    deg = jax.ops.segment_sum(jnp.ones((_E,), jnp.float32), dst,
                              num_segments=_N)[:, None]

    hh1a, hh1b = _nodelin(x, c1_lin_w, c1_lin_b.reshape(1, _H))
    s1a, s1b = _gather_scatter(hh1a, hh1b, src, dst)
    a1a, a1b = _scatter_rows(gf1a, gf1b, dst)
    a2a, a2b = _scatter_rows(gf2a, gf2b, dst)

    hh2a, hh2b = _combine_lin(s1a, s1b, a1a, a1b, deg, c1_g2_w,
                              c1_g2_b.reshape(1, _H), c2_lin_w,
                              c2_lin_b.reshape(1, _H))
    s2a, s2b = _gather_scatter(hh2a, hh2b, src, dst)

    onehot = (batch[None, :].astype(jnp.int32)
              == jnp.arange(_NG, dtype=jnp.int32)[:, None]).astype(jnp.float32)
    l2wp = jnp.pad(l2_w, ((0, 0), (0, _H - 1)))
    l2bp = jnp.pad(l2_b, (0, _H - 1)).reshape(1, _H)
    return _final_head(s2a, s2b, a2a, a2b, deg, c2_g2_w,
                       c2_g2_b.reshape(1, _H), sa_w, sa_b.reshape(1, _H),
                       onehot, l1_w, l1_b.reshape(1, 64), l2wp, l2bp)


# speedup vs baseline: 10.1370x; 10.1370x over previous
import functools
import jax
import jax.numpy as jnp
from jax import lax
from jax.experimental import pallas as pl
from jax.experimental.pallas import tpu as pltpu
from jax.experimental.pallas import tpu_sc as plsc

_N = 10000
_E = 320000
_EPAD = 327680
_H = 128
_NG = 64
_EPS = 1e-8
_BIG = 1e30


def _trig_gf_body(vd_ref, w1_ref, b1_ref, w2_ref, b2_ref, o1a, o1b, o2a, o2b):
    vd = vd_ref[...]
    v1x, v1y, v1z = vd[:, 0:1], vd[:, 1:2], vd[:, 2:3]
    v2x, v2y, v2z = vd[:, 3:4], vd[:, 4:5], vd[:, 5:6]
    v3x, v3y, v3z = vd[:, 6:7], vd[:, 7:8], vd[:, 8:9]
    d2 = vd[:, 9:10]
    d = jnp.sqrt(d2)
    ct = jnp.clip(v2z / (d + _EPS), -1.0 + _EPS, 1.0 - _EPS)
    theta = jnp.arctan2(jnp.sqrt(jnp.maximum(1.0 - ct * ct, 0.0)), ct)
    phi = jnp.arctan2(v2y, v2x)
    n1x = v1y * v2z - v1z * v2y
    n1y = v1z * v2x - v1x * v2z
    n1z = v1x * v2y - v1y * v2x
    n2x = v2y * v3z - v2z * v3y
    n2y = v2z * v3x - v2x * v3z
    n2z = v2x * v3y - v2y * v3x
    dotn = n1x * n2x + n1y * n2y + n1z * n2z
    nn1 = jnp.sqrt(n1x * n1x + n1y * n1y + n1z * n1z) + _EPS
    nn2 = jnp.sqrt(n2x * n2x + n2y * n2y + n2z * n2z) + _EPS
    ctau = jnp.clip(dotn / (nn1 * nn2), -1.0 + _EPS, 1.0 - _EPS)
    tau = jnp.arctan2(jnp.sqrt(jnp.maximum(1.0 - ctau * ctau, 0.0)), ctau)

    def gf(w_ref, b_ref):
        w = w_ref[...]
        return jnp.maximum(d * w[0:1, :] + theta * w[1:2, :] + phi * w[2:3, :]
                           + tau * w[3:4, :] + b_ref[...], 0.0)

    g1 = gf(w1_ref, b1_ref)
    g2 = gf(w2_ref, b2_ref)
    o1a[...] = g1[:, :64]
    o1b[...] = g1[:, 64:]
    o2a[...] = g2[:, :64]
    o2b[...] = g2[:, 64:]


def _trig_gf(vd, w1, b1, w2, b2):
    be = 4096
    grid = (_EPAD // be,)
    half = jax.ShapeDtypeStruct((_EPAD, 64), jnp.float32)
    return pl.pallas_call(
        _trig_gf_body,
        grid=grid,
        in_specs=[
            pl.BlockSpec((be, 16), lambda i: (i, 0)),
            pl.BlockSpec((4, _H), lambda i: (0, 0)),
            pl.BlockSpec((1, _H), lambda i: (0, 0)),
            pl.BlockSpec((4, _H), lambda i: (0, 0)),
            pl.BlockSpec((1, _H), lambda i: (0, 0)),
        ],
        out_specs=[pl.BlockSpec((be, 64), lambda i: (i, 0))] * 4,
        out_shape=[half, half, half, half],
    )(vd, w1, b1, w2, b2)


def _nodelin_body(h_ref, w_ref, b_ref, oa, ob):
    hh = jnp.dot(h_ref[...], w_ref[...],
                 preferred_element_type=jnp.float32) + b_ref[...]
    oa[...] = hh[:, :64]
    ob[...] = hh[:, 64:]


def _nodelin(h, w, b):
    bn = 2000
    half = jax.ShapeDtypeStruct((_N, 64), jnp.float32)
    return pl.pallas_call(
        _nodelin_body,
        grid=(_N // bn,),
        in_specs=[
            pl.BlockSpec((bn, _H), lambda i: (i, 0)),
            pl.BlockSpec((_H, _H), lambda i: (0, 0)),
            pl.BlockSpec((1, _H), lambda i: (0, 0)),
        ],
        out_specs=[pl.BlockSpec((bn, 64), lambda i: (i, 0))] * 2,
        out_shape=[half, half],
    )(h, w, b)


def _combine_body(s1a, s1b, aa, ab, deg_ref, g2wa, g2wb, g2b, lw, lb, oa, ob):
    acont = (jnp.dot(aa[...], g2wa[...], preferred_element_type=jnp.float32)
             + jnp.dot(ab[...], g2wb[...], preferred_element_type=jnp.float32))
    s1 = jnp.concatenate([s1a[...], s1b[...]], axis=1)
    h = jnp.maximum(s1 + acont + deg_ref[...] * g2b[...], 0.0)
    hh = jnp.dot(h, lw[...], preferred_element_type=jnp.float32) + lb[...]
    oa[...] = hh[:, :64]
    ob[...] = hh[:, 64:]


def _combine_lin(s1a, s1b, aa, ab, deg, g2w, g2b, lw, lb):
    bn = 2000
    half = jax.ShapeDtypeStruct((_N, 64), jnp.float32)
    return pl.pallas_call(
        _combine_body,
        grid=(_N // bn,),
        in_specs=[
            pl.BlockSpec((bn, 64), lambda i: (i, 0)),
            pl.BlockSpec((bn, 64), lambda i: (i, 0)),
            pl.BlockSpec((bn, 64), lambda i: (i, 0)),
            pl.BlockSpec((bn, 64), lambda i: (i, 0)),
            pl.BlockSpec((bn, 1), lambda i: (i, 0)),
            pl.BlockSpec((64, _H), lambda i: (0, 0)),
            pl.BlockSpec((64, _H), lambda i: (0, 0)),
            pl.BlockSpec((1, _H), lambda i: (0, 0)),
            pl.BlockSpec((_H, _H), lambda i: (0, 0)),
            pl.BlockSpec((1, _H), lambda i: (0, 0)),
        ],
        out_specs=[pl.BlockSpec((bn, 64), lambda i: (i, 0))] * 2,
        out_shape=[half, half],
    )(s1a, s1b, aa, ab, deg, g2w[:64], g2w[64:], g2b, lw, lb)


def _final_body(s1a, s1b, aa, ab, deg_ref, g2wa, g2wb, g2b, saw, sab,
                oh_ref, l1w, l1b, l2w, l2b, sums_ref, cnt_ref, out_ref):
    i = pl.program_id(0)
    nsteps = pl.num_programs(0)
    acont = (jnp.dot(aa[...], g2wa[...], preferred_element_type=jnp.float32)
             + jnp.dot(ab[...], g2wb[...], preferred_element_type=jnp.float32))
    s1 = jnp.concatenate([s1a[...], s1b[...]], axis=1)
    h2 = jnp.maximum(s1 + acont + deg_ref[...] * g2b[...], 0.0)
    h3 = jnp.maximum(jnp.dot(h2, saw[...],
                             preferred_element_type=jnp.float32) + sab[...], 0.0)
    oh = oh_ref[...]
    psum = jnp.dot(oh, h3, preferred_element_type=jnp.float32)
    ones = jnp.ones((oh.shape[1], _H), jnp.float32)
    pcnt = jnp.dot(oh, ones, preferred_element_type=jnp.float32)

    @pl.when(i == 0)
    def _():
        sums_ref[...] = psum
        cnt_ref[...] = pcnt

    @pl.when(i > 0)
    def _():
        sums_ref[...] += psum
        cnt_ref[...] += pcnt

    @pl.when(i == nsteps - 1)
    def _():
        cnt = cnt_ref[...]
        sums = sums_ref[...]
        pooled = jnp.where(cnt > 0, sums / jnp.maximum(cnt, 1.0), 0.0)
        hf = jnp.maximum(jnp.dot(pooled, l1w[...],
                                 preferred_element_type=jnp.float32) + l1b[...], 0.0)
        out_ref[...] = jnp.dot(hf, l2w[...],
                               preferred_element_type=jnp.float32) + l2b[...]


def _final_head(s1a, s1b, aa, ab, deg, g2w, g2b, saw, sab, onehot,
                l1w, l1b, l2wp, l2bp):
    bn = 2048
    npad = 10240
    pad = lambda t: jnp.pad(t, ((0, npad - _N), (0, 0)))
    s1a, s1b, aa, ab, deg = map(pad, (s1a, s1b, aa, ab, deg))
    onehot = jnp.pad(onehot, ((0, 0), (0, npad - _N)))
    outs = [jax.ShapeDtypeStruct((_NG, _H), jnp.float32),
            jax.ShapeDtypeStruct((_NG, _H), jnp.float32),
            jax.ShapeDtypeStruct((_NG, _H), jnp.float32)]
    sums, cnt, out = pl.pallas_call(
        _final_body,
        grid=(npad // bn,),
        in_specs=[
            pl.BlockSpec((bn, 64), lambda i: (i, 0)),
            pl.BlockSpec((bn, 64), lambda i: (i, 0)),
            pl.BlockSpec((bn, 64), lambda i: (i, 0)),
            pl.BlockSpec((bn, 64), lambda i: (i, 0)),
            pl.BlockSpec((bn, 1), lambda i: (i, 0)),
            pl.BlockSpec((64, _H), lambda i: (0, 0)),
            pl.BlockSpec((64, _H), lambda i: (0, 0)),
            pl.BlockSpec((1, _H), lambda i: (0, 0)),
            pl.BlockSpec((_H, _H), lambda i: (0, 0)),
            pl.BlockSpec((1, _H), lambda i: (0, 0)),
            pl.BlockSpec((_NG, bn), lambda i: (0, i)),
            pl.BlockSpec((_H, 64), lambda i: (0, 0)),
            pl.BlockSpec((1, 64), lambda i: (0, 0)),
            pl.BlockSpec((64, _H), lambda i: (0, 0)),
            pl.BlockSpec((1, _H), lambda i: (0, 0)),
        ],
        out_specs=[pl.BlockSpec((_NG, _H), lambda i: (0, 0))] * 3,
        out_shape=outs,
    )(s1a, s1b, aa, ab, deg, g2w[:64], g2w[64:], g2b, saw, sab, onehot,
      l1w, l1b, l2wp, l2bp)
    return out[:, :1]


_NPAD = 10240
_CH = 80
_NW = 32
_EPW = _E // _NW
_NCHUNK = _EPW // _CH


_SEG = 2000
_NSEG = _EPW // _SEG
_GRP = _SEG // 16
_SCPARAMS = pltpu.CompilerParams(use_tc_tiling_on_sc=False,
                                 needs_layout_passes=False)


def _rmw_min(tbl_v, idx16, val16, act16):
    t0 = plsc.load_gather(tbl_v, [idx16])
    m0 = act16 & (val16 < t0)
    cnt0 = jnp.sum(m0.astype(jnp.int32))

    def cond(cnt):
        return cnt > 0

    def body(cnt):
        t = plsc.load_gather(tbl_v, [idx16])
        m = act16 & (val16 < t)
        plsc.store_scatter(tbl_v, [idx16], val16, mask=m)
        t2 = plsc.load_gather(tbl_v, [idx16])
        m2 = act16 & (val16 < t2)
        return jnp.sum(m2.astype(jnp.int32))

    lax.while_loop(cond, body, cnt0)


def _tile_combine(tbl_v, sh_tbl, rbuf, racc, out_h, c, s, kind):
    zsl = _NPAD // 16
    pltpu.sync_copy(tbl_v, sh_tbl.at[s])
    plsc.subcore_barrier()
    pltpu.sync_copy(sh_tbl.at[0, pl.ds(s * zsl, zsl)], racc)
    for r in range(1, 16):
        pltpu.sync_copy(sh_tbl.at[r, pl.ds(s * zsl, zsl)], rbuf)

        def red(q, _, r=r):
            a = racc[pl.ds(q * 16, 16)]
            b = rbuf[pl.ds(q * 16, 16)]
            racc[pl.ds(q * 16, 16)] = (jnp.minimum(a, b) if kind == "min"
                                       else a + b)
            return 0

        lax.fori_loop(0, zsl // 16, red, 0)
    pltpu.sync_copy(racc, out_h.at[pl.ds(c * _NPAD + s * zsl, zsl)])


def _sc_geo1(posx, posy, posz, srcf, dstf, bigf):
    mesh = plsc.VectorSubcoreMesh(core_axis_name="c", subcore_axis_name="s")
    out_type = [jax.ShapeDtypeStruct((_E,), jnp.float32),
                jax.ShapeDtypeStruct((2 * _NPAD,), jnp.float32),
                jax.ShapeDtypeStruct((2 * _NPAD,), jnp.float32)]
    scratch = [pltpu.VMEM((_NPAD,), jnp.float32)] * 3 \
        + [pltpu.VMEM((_SEG,), jnp.int32)] * 2 \
        + [pltpu.VMEM((_SEG,), jnp.float32)] \
        + [pltpu.VMEM((_NPAD,), jnp.float32)] * 2 \
        + [pltpu.VMEM_SHARED((16, _NPAD), jnp.float32)] \
        + [pltpu.VMEM((_NPAD // 16,), jnp.float32)] * 2

    @functools.partial(pl.kernel, mesh=mesh, out_type=out_type,
                       scratch_types=scratch, compiler_params=_SCPARAMS)
    def k(posx_h, posy_h, posz_h, src_h, dst_h, big_h, d2_o, min1_o, deg_o,
          px, py, pz, sb, db, d2b, tmin, tdeg, sh_tbl, rbuf, racc):
        c = lax.axis_index("c")
        s = lax.axis_index("s")
        w = s * 2 + c
        pltpu.sync_copy(posx_h, px)
        pltpu.sync_copy(posy_h, py)
        pltpu.sync_copy(posz_h, pz)
        pltpu.sync_copy(big_h, tmin)
        ones16 = jnp.full((16,), 1.0, jnp.float32)

        def zero(q, _):
            tdeg[pl.ds(q * 16, 16)] = jnp.zeros((16,), jnp.float32)
            return 0

        lax.fori_loop(0, _NPAD // 16, zero, 0)
        true16 = jnp.full((16,), True)
        for seg in range(_NSEG):
            base = w * _EPW + seg * _SEG
            pltpu.sync_copy(src_h.at[pl.ds(base, _SEG)], sb)
            pltpu.sync_copy(dst_h.at[pl.ds(base, _SEG)], db)

            def grp(i, _):
                s16 = sb[pl.ds(i * 16, 16)]
                t16 = db[pl.ds(i * 16, 16)]
                dx = (plsc.load_gather(px, [t16])
                      - plsc.load_gather(px, [s16]))
                dy = (plsc.load_gather(py, [t16])
                      - plsc.load_gather(py, [s16]))
                dz = (plsc.load_gather(pz, [t16])
                      - plsc.load_gather(pz, [s16]))
                d2 = dx * dx + dy * dy + dz * dz
                d2b[pl.ds(i * 16, 16)] = d2
                _rmw_min(tmin, s16, d2, true16)
                plsc.addupdate_scatter(tdeg, [t16], ones16)
                return 0

            lax.fori_loop(0, _GRP, grp, 0)
            pltpu.sync_copy(d2b, d2_o.at[pl.ds(base, _SEG)])
        _tile_combine(tmin, sh_tbl, rbuf, racc, min1_o, c, s, "min")
        plsc.subcore_barrier()
        _tile_combine(tdeg, sh_tbl, rbuf, racc, deg_o, c, s, "sum")

    return k(posx, posy, posz, srcf, dstf, bigf)


def _sc_rmw_pass(mode, d2e, srcf, dstf, init_tbl, glbs):
    tdt = jnp.int32 if mode in ("nn1", "nn2") else jnp.float32
    gdts = {"nn1": [jnp.float32], "min2": [jnp.int32],
            "nn2": [jnp.int32, jnp.float32]}[mode]
    nglb = len(glbs)
    mesh = plsc.VectorSubcoreMesh(core_axis_name="c", subcore_axis_name="s")
    out_type = jax.ShapeDtypeStruct((2 * _NPAD,), tdt)
    scratch = ([pltpu.VMEM((_SEG,), jnp.int32)] * 2
               + [pltpu.VMEM((_SEG,), jnp.float32)]
               + [pltpu.VMEM((_NPAD,), g) for g in gdts]
               + [pltpu.VMEM((_NPAD,), tdt)]
               + [pltpu.VMEM_SHARED((16, _NPAD), tdt)]
               + [pltpu.VMEM((_NPAD // 16,), tdt)] * 2)

    @functools.partial(pl.kernel, mesh=mesh, out_type=out_type,
                       scratch_types=scratch, compiler_params=_SCPARAMS)
    def k(*refs):
        d2_h, src_h, dst_h, init_h = refs[:4]
        glb_h = refs[4:4 + nglb]
        out_o = refs[4 + nglb]
        sb, db, d2b = refs[5 + nglb:8 + nglb]
        glb_v = refs[8 + nglb:8 + 2 * nglb]
        tloc = refs[8 + 2 * nglb]
        sh_tbl, rbuf, racc = refs[9 + 2 * nglb:12 + 2 * nglb]
        c = lax.axis_index("c")
        s = lax.axis_index("s")
        w = s * 2 + c
        for gh, gv in zip(glb_h, glb_v):
            pltpu.sync_copy(gh, gv)
        pltpu.sync_copy(init_h, tloc)
        for seg in range(_NSEG):
            base = w * _EPW + seg * _SEG
            pltpu.sync_copy(src_h.at[pl.ds(base, _SEG)], sb)
            pltpu.sync_copy(dst_h.at[pl.ds(base, _SEG)], db)
            pltpu.sync_copy(d2_h.at[pl.ds(base, _SEG)], d2b)

            def grp(i, _):
                s16 = sb[pl.ds(i * 16, 16)]
                t16 = db[pl.ds(i * 16, 16)]
                d216 = d2b[pl.ds(i * 16, 16)]
                if mode == "nn1":
                    act = d216 == plsc.load_gather(glb_v[0], [s16])
                    val = t16
                elif mode == "min2":
                    act = plsc.load_gather(glb_v[0], [s16]) != t16
                    val = d216
                else:
                    act = ((plsc.load_gather(glb_v[0], [s16]) != t16)
                           & (d216 == plsc.load_gather(glb_v[1], [s16])))
                    val = t16
                _rmw_min(tloc, s16, val, act)
                return 0

            lax.fori_loop(0, _GRP, grp, 0)
        _tile_combine(tloc, sh_tbl, rbuf, racc, out_o, c, s, "min")

    return k(d2e, srcf, dstf, init_tbl, *glbs)


def _sc_geo5(posx, posy, posz, srcf, dstf, nn1, nn2, min1, min2):
    mesh = plsc.VectorSubcoreMesh(core_axis_name="c", subcore_axis_name="s")
    out_type = jax.ShapeDtypeStruct((_EPAD, 16), jnp.float32)
    scratch = ([pltpu.VMEM((_NPAD,), jnp.float32)] * 3
               + [pltpu.VMEM((_NPAD,), jnp.int32)] * 2
               + [pltpu.VMEM((_NPAD,), jnp.float32)] * 2
               + [pltpu.VMEM((_SEG,), jnp.int32)] * 2
               + [pltpu.VMEM((_SEG, 16), jnp.float32)])

    @functools.partial(pl.kernel, mesh=mesh, out_type=out_type,
                       scratch_types=scratch, compiler_params=_SCPARAMS)
    def k(posx_h, posy_h, posz_h, src_h, dst_h, nn1_h, nn2_h, m1_h, m2_h,
          vd_o, px, py, pz, n1v, n2v, m1v, m2v, sb, db, stag):
        c = lax.axis_index("c")
        s = lax.axis_index("s")
        w = s * 2 + c
        pltpu.sync_copy(posx_h, px)
        pltpu.sync_copy(posy_h, py)
        pltpu.sync_copy(posz_h, pz)
        pltpu.sync_copy(nn1_h, n1v)
        pltpu.sync_copy(nn2_h, n2v)
        pltpu.sync_copy(m1_h, m1v)
        pltpu.sync_copy(m2_h, m2v)
        lanes = lax.iota(jnp.int32, 16)
        for seg in range(_NSEG):
            base = w * _EPW + seg * _SEG
            pltpu.sync_copy(src_h.at[pl.ds(base, _SEG)], sb)
            pltpu.sync_copy(dst_h.at[pl.ds(base, _SEG)], db)

            def grp(i, _):
                s16 = sb[pl.ds(i * 16, 16)]
                t16 = db[pl.ds(i * 16, 16)]
                lg = plsc.load_gather
                n1s, n2s = lg(n1v, [s16]), lg(n2v, [s16])
                h1s, h2s = lg(m1v, [s16]) < _BIG, lg(m2v, [s16]) < _BIG
                n1d, n2d = lg(n1v, [t16]), lg(n2v, [t16])
                h1d, h2d = lg(m1v, [t16]) < _BIG, lg(m2v, [t16]) < _BIG
                fi = jnp.where(~h1s, s16,
                               jnp.where(n1s != t16, n1s,
                                         jnp.where(h2s, n2s, s16)))
                fj = jnp.where(~h1d, t16,
                               jnp.where(n1d != s16, n1d,
                                         jnp.where(h2d, n2d, t16)))
                pxs, pys, pzs = lg(px, [s16]), lg(py, [s16]), lg(pz, [s16])
                pxt, pyt, pzt = lg(px, [t16]), lg(py, [t16]), lg(pz, [t16])
                v1 = (pxs - lg(px, [fi]), pys - lg(py, [fi]),
                      pzs - lg(pz, [fi]))
                v2 = (pxt - pxs, pyt - pys, pzt - pzs)
                v3 = (pxt - lg(px, [fj]), pyt - lg(py, [fj]),
                      pzt - lg(pz, [fj]))
                d2 = v2[0] * v2[0] + v2[1] * v2[1] + v2[2] * v2[2]
                rows16 = lanes + i * 16
                cols = v1 + v2 + v3 + (d2,)
                for colid, cv in enumerate(cols):
                    plsc.store_scatter(
                        stag, [rows16, jnp.full((16,), colid, jnp.int32)], cv)
                return 0

            lax.fori_loop(0, _GRP, grp, 0)
            pltpu.sync_copy(stag, vd_o.at[pl.ds(base, _SEG)])

    return k(posx, posy, posz, srcf, dstf, nn1, nn2, min1, min2)


def _sc_agg_call(tbl, gfs, src3, dst3, zeros):
    ngf = len(gfs)
    ntbl = 0 if tbl is None else 1
    nacc = ntbl + ngf
    assert nacc <= 2
    mesh = plsc.VectorSubcoreMesh(core_axis_name="c", subcore_axis_name="s")
    out_type = [jax.ShapeDtypeStruct((2 * _NPAD, 64), jnp.float32)] * nacc
    scratch = ([pltpu.VMEM((_NCHUNK, _CH), jnp.int32),
                pltpu.VMEM((_NCHUNK, _CH), jnp.int32)]
               + [pltpu.VMEM((_CH, 64), jnp.float32)] * nacc
               + [pltpu.VMEM_SHARED((_NPAD, 64), jnp.float32)] * nacc
               + [pltpu.SemaphoreType.DMA])

    @functools.partial(pl.kernel, mesh=mesh, out_type=out_type,
                       scratch_types=scratch, compiler_params=_SCPARAMS)
    def k(*refs):
        tbl_h = refs[0] if ntbl else None
        rest = refs[ntbl:]
        gf_h = rest[:ngf]
        s3, d3, zer_h = rest[ngf:ngf + 3]
        outs = rest[ngf + 3:ngf + 3 + nacc]
        sidx, didx = rest[ngf + 3 + nacc:ngf + 5 + nacc]
        bufs = rest[ngf + 5 + nacc:ngf + 5 + 2 * nacc]
        accs = rest[ngf + 5 + 2 * nacc:ngf + 5 + 3 * nacc]
        sem = rest[-1]
        c = lax.axis_index("c")
        s = lax.axis_index("s")
        w = s * 2 + c
        zslice = _NPAD // 16
        for acc in accs:
            pltpu.sync_copy(zer_h.at[pl.ds(s * zslice, zslice)],
                            acc.at[pl.ds(s * zslice, zslice)])
        pltpu.sync_copy(s3.at[w], sidx)
        pltpu.sync_copy(d3.at[w], didx)
        plsc.subcore_barrier()

        def step(j, carry):
            di = didx.at[j]
            if ntbl:
                pltpu.async_copy(tbl_h.at[sidx.at[j]], bufs[0], sem).wait()
            for g in range(ngf):
                pltpu.sync_copy(gf_h[g].at[w * _NCHUNK + j], bufs[ntbl + g])
            for b, acc in zip(bufs, accs):
                pltpu.sync_copy(b, acc.at[di], add=True)
            return carry

        lax.fori_loop(0, _NCHUNK, step, 0)
        plsc.subcore_barrier()
        for acc, out in zip(accs, outs):
            pltpu.sync_copy(acc.at[pl.ds(s * zslice, zslice)],
                            out.at[pl.ds(c * _NPAD + s * zslice, zslice)])

    args = ([] if tbl is None else [tbl]) + list(gfs) + [src3, dst3, zeros]
    res = k(*args)
    res = res if isinstance(res, (list, tuple)) else [res]
    return [r[:_NPAD] + r[_NPAD:] for r in res]


def kernel(x, edge_index, batch, pos, c1_lin_w, c1_lin_b, c1_g1_w, c1_g1_b,
           c1_g2_w, c1_g2_b, c2_lin_w, c2_lin_b, c2_g1_w, c2_g1_b, c2_g2_w,
           c2_g2_b, sa_w, sa_b, l1_w, l1_b, l2_w, l2_b):
    src = edge_index[0].astype(jnp.int32)
    dst = edge_index[1].astype(jnp.int32)

    posp = jnp.pad(pos, ((0, _NPAD - _N), (0, 0)))
    posx, posy, posz = posp[:, 0], posp[:, 1], posp[:, 2]
    bigf = jnp.full((_NPAD,), _BIG, jnp.float32)
    sentn = jnp.full((_NPAD,), _N, jnp.int32)
    d2e, min1p, degp = _sc_geo1(posx, posy, posz, src, dst, bigf)
    min1 = jnp.minimum(min1p[:_NPAD], min1p[_NPAD:])
    deg = (degp[:_NPAD] + degp[_NPAD:])[:_N, None]
    nn1p = _sc_rmw_pass("nn1", d2e, src, dst, sentn, [min1])
    nn1 = jnp.minimum(nn1p[:_NPAD], nn1p[_NPAD:])
    min2p = _sc_rmw_pass("min2", d2e, src, dst, bigf, [nn1])
    min2 = jnp.minimum(min2p[:_NPAD], min2p[_NPAD:])
    nn2p = _sc_rmw_pass("nn2", d2e, src, dst, sentn, [nn1, min2])
    nn2 = jnp.minimum(nn2p[:_NPAD], nn2p[_NPAD:])
    vd = _sc_geo5(posx, posy, posz, src, dst, nn1, nn2, min1, min2)

    gf1a, gf1b, gf2a, gf2b = _trig_gf(
        vd, c1_g1_w, c1_g1_b.reshape(1, _H), c2_g1_w, c2_g1_b.reshape(1, _H))

    src3 = src.reshape(_NW, _NCHUNK, _CH)
    dst3 = dst.reshape(_NW, _NCHUNK, _CH)
    zeros = jnp.zeros((_NPAD, 64), jnp.float32)
    rs = lambda g: g[:_E].reshape(_E // _CH, _CH, 64)

    hh1a, hh1b = _nodelin(x, c1_lin_w, c1_lin_b.reshape(1, _H))
    s1a, a1a = _sc_agg_call(hh1a, [rs(gf1a)], src3, dst3, zeros)
    s1b, a2a = _sc_agg_call(hh1b, [rs(gf2a)], src3, dst3, zeros)
    a1b, a2b = _sc_agg_call(None, [rs(gf1b), rs(gf2b)], src3, dst3, zeros)
    s1a, s1b, a1a, a1b = s1a[:_N], s1b[:_N], a1a[:_N], a1b[:_N]

    hh2a, hh2b = _combine_lin(s1a, s1b, a1a, a1b, deg, c1_g2_w,
                              c1_g2_b.reshape(1, _H), c2_lin_w,
                              c2_lin_b.reshape(1, _H))
    (s2a,) = _sc_agg_call(hh2a, [], src3, dst3, zeros)
    (s2b,) = _sc_agg_call(hh2b, [], src3, dst3, zeros)
    s2a, s2b, a2a, a2b = s2a[:_N], s2b[:_N], a2a[:_N], a2b[:_N]

    onehot = (batch[None, :].astype(jnp.int32)
              == jnp.arange(_NG, dtype=jnp.int32)[:, None]).astype(jnp.float32)
    l2wp = jnp.pad(l2_w, ((0, 0), (0, _H - 1)))
    l2bp = jnp.pad(l2_b, (0, _H - 1)).reshape(1, _H)
    return _final_head(s2a, s2b, a2a, a2b, deg, c2_g2_w,
                       c2_g2_b.reshape(1, _H), sa_w, sa_b.reshape(1, _H),
                       onehot, l1_w, l1_b.reshape(1, 64), l2wp, l2bp)


# speedup vs baseline: 11.5816x; 1.1425x over previous
import functools
import jax
import jax.numpy as jnp
from jax import lax
from jax.experimental import pallas as pl
from jax.experimental.pallas import tpu as pltpu
from jax.experimental.pallas import tpu_sc as plsc

_N = 10000
_E = 320000
_EPAD = 327680
_H = 128
_NG = 64
_EPS = 1e-8
_BIG = 1e30


def _trig_gf_body(vd_ref, w1_ref, b1_ref, w2_ref, b2_ref, o1a, o1b, o2a, o2b):
    vd = vd_ref[...]
    v1x, v1y, v1z = vd[:, 0:1], vd[:, 1:2], vd[:, 2:3]
    v2x, v2y, v2z = vd[:, 3:4], vd[:, 4:5], vd[:, 5:6]
    v3x, v3y, v3z = vd[:, 6:7], vd[:, 7:8], vd[:, 8:9]
    d2 = vd[:, 9:10]
    d = jnp.sqrt(d2)
    ct = jnp.clip(v2z / (d + _EPS), -1.0 + _EPS, 1.0 - _EPS)
    theta = jnp.arctan2(jnp.sqrt(jnp.maximum(1.0 - ct * ct, 0.0)), ct)
    phi = jnp.arctan2(v2y, v2x)
    n1x = v1y * v2z - v1z * v2y
    n1y = v1z * v2x - v1x * v2z
    n1z = v1x * v2y - v1y * v2x
    n2x = v2y * v3z - v2z * v3y
    n2y = v2z * v3x - v2x * v3z
    n2z = v2x * v3y - v2y * v3x
    dotn = n1x * n2x + n1y * n2y + n1z * n2z
    nn1 = jnp.sqrt(n1x * n1x + n1y * n1y + n1z * n1z) + _EPS
    nn2 = jnp.sqrt(n2x * n2x + n2y * n2y + n2z * n2z) + _EPS
    ctau = jnp.clip(dotn / (nn1 * nn2), -1.0 + _EPS, 1.0 - _EPS)
    tau = jnp.arctan2(jnp.sqrt(jnp.maximum(1.0 - ctau * ctau, 0.0)), ctau)

    def gf(w_ref, b_ref):
        w = w_ref[...]
        return jnp.maximum(d * w[0:1, :] + theta * w[1:2, :] + phi * w[2:3, :]
                           + tau * w[3:4, :] + b_ref[...], 0.0)

    g1 = gf(w1_ref, b1_ref)
    g2 = gf(w2_ref, b2_ref)
    o1a[...] = g1[:, :64]
    o1b[...] = g1[:, 64:]
    o2a[...] = g2[:, :64]
    o2b[...] = g2[:, 64:]


def _trig_gf(vd, w1, b1, w2, b2):
    be = 4096
    grid = (_EPAD // be,)
    half = jax.ShapeDtypeStruct((_EPAD, 64), jnp.float32)
    return pl.pallas_call(
        _trig_gf_body,
        grid=grid,
        in_specs=[
            pl.BlockSpec((be, 16), lambda i: (i, 0)),
            pl.BlockSpec((4, _H), lambda i: (0, 0)),
            pl.BlockSpec((1, _H), lambda i: (0, 0)),
            pl.BlockSpec((4, _H), lambda i: (0, 0)),
            pl.BlockSpec((1, _H), lambda i: (0, 0)),
        ],
        out_specs=[pl.BlockSpec((be, 64), lambda i: (i, 0))] * 4,
        out_shape=[half, half, half, half],
    )(vd, w1, b1, w2, b2)


def _nodelin_body(h_ref, w_ref, b_ref, oa, ob):
    hh = jnp.dot(h_ref[...], w_ref[...],
                 preferred_element_type=jnp.float32) + b_ref[...]
    oa[...] = hh[:, :64]
    ob[...] = hh[:, 64:]


def _nodelin(h, w, b):
    bn = 2000
    half = jax.ShapeDtypeStruct((_N, 64), jnp.float32)
    return pl.pallas_call(
        _nodelin_body,
        grid=(_N // bn,),
        in_specs=[
            pl.BlockSpec((bn, _H), lambda i: (i, 0)),
            pl.BlockSpec((_H, _H), lambda i: (0, 0)),
            pl.BlockSpec((1, _H), lambda i: (0, 0)),
        ],
        out_specs=[pl.BlockSpec((bn, 64), lambda i: (i, 0))] * 2,
        out_shape=[half, half],
    )(h, w, b)


def _combine_body(s1a, s1b, aa, ab, deg_ref, g2wa, g2wb, g2b, lw, lb, oa, ob):
    acont = (jnp.dot(aa[...], g2wa[...], preferred_element_type=jnp.float32)
             + jnp.dot(ab[...], g2wb[...], preferred_element_type=jnp.float32))
    s1 = jnp.concatenate([s1a[...], s1b[...]], axis=1)
    h = jnp.maximum(s1 + acont + deg_ref[...] * g2b[...], 0.0)
    hh = jnp.dot(h, lw[...], preferred_element_type=jnp.float32) + lb[...]
    oa[...] = hh[:, :64]
    ob[...] = hh[:, 64:]


def _combine_lin(s1a, s1b, aa, ab, deg, g2w, g2b, lw, lb):
    bn = 2000
    half = jax.ShapeDtypeStruct((_N, 64), jnp.float32)
    return pl.pallas_call(
        _combine_body,
        grid=(_N // bn,),
        in_specs=[
            pl.BlockSpec((bn, 64), lambda i: (i, 0)),
            pl.BlockSpec((bn, 64), lambda i: (i, 0)),
            pl.BlockSpec((bn, 64), lambda i: (i, 0)),
            pl.BlockSpec((bn, 64), lambda i: (i, 0)),
            pl.BlockSpec((bn, 1), lambda i: (i, 0)),
            pl.BlockSpec((64, _H), lambda i: (0, 0)),
            pl.BlockSpec((64, _H), lambda i: (0, 0)),
            pl.BlockSpec((1, _H), lambda i: (0, 0)),
            pl.BlockSpec((_H, _H), lambda i: (0, 0)),
            pl.BlockSpec((1, _H), lambda i: (0, 0)),
        ],
        out_specs=[pl.BlockSpec((bn, 64), lambda i: (i, 0))] * 2,
        out_shape=[half, half],
    )(s1a, s1b, aa, ab, deg, g2w[:64], g2w[64:], g2b, lw, lb)


def _final_body(s1a, s1b, aa, ab, deg_ref, g2wa, g2wb, g2b, saw, sab,
                oh_ref, l1w, l1b, l2w, l2b, sums_ref, cnt_ref, out_ref):
    i = pl.program_id(0)
    nsteps = pl.num_programs(0)
    acont = (jnp.dot(aa[...], g2wa[...], preferred_element_type=jnp.float32)
             + jnp.dot(ab[...], g2wb[...], preferred_element_type=jnp.float32))
    s1 = jnp.concatenate([s1a[...], s1b[...]], axis=1)
    h2 = jnp.maximum(s1 + acont + deg_ref[...] * g2b[...], 0.0)
    h3 = jnp.maximum(jnp.dot(h2, saw[...],
                             preferred_element_type=jnp.float32) + sab[...], 0.0)
    oh = oh_ref[...]
    psum = jnp.dot(oh, h3, preferred_element_type=jnp.float32)
    ones = jnp.ones((oh.shape[1], _H), jnp.float32)
    pcnt = jnp.dot(oh, ones, preferred_element_type=jnp.float32)

    @pl.when(i == 0)
    def _():
        sums_ref[...] = psum
        cnt_ref[...] = pcnt

    @pl.when(i > 0)
    def _():
        sums_ref[...] += psum
        cnt_ref[...] += pcnt

    @pl.when(i == nsteps - 1)
    def _():
        cnt = cnt_ref[...]
        sums = sums_ref[...]
        pooled = jnp.where(cnt > 0, sums / jnp.maximum(cnt, 1.0), 0.0)
        hf = jnp.maximum(jnp.dot(pooled, l1w[...],
                                 preferred_element_type=jnp.float32) + l1b[...], 0.0)
        out_ref[...] = jnp.dot(hf, l2w[...],
                               preferred_element_type=jnp.float32) + l2b[...]


def _final_head(s1a, s1b, aa, ab, deg, g2w, g2b, saw, sab, onehot,
                l1w, l1b, l2wp, l2bp):
    bn = 2048
    npad = 10240
    pad = lambda t: jnp.pad(t, ((0, npad - _N), (0, 0)))
    s1a, s1b, aa, ab, deg = map(pad, (s1a, s1b, aa, ab, deg))
    onehot = jnp.pad(onehot, ((0, 0), (0, npad - _N)))
    outs = [jax.ShapeDtypeStruct((_NG, _H), jnp.float32),
            jax.ShapeDtypeStruct((_NG, _H), jnp.float32),
            jax.ShapeDtypeStruct((_NG, _H), jnp.float32)]
    sums, cnt, out = pl.pallas_call(
        _final_body,
        grid=(npad // bn,),
        in_specs=[
            pl.BlockSpec((bn, 64), lambda i: (i, 0)),
            pl.BlockSpec((bn, 64), lambda i: (i, 0)),
            pl.BlockSpec((bn, 64), lambda i: (i, 0)),
            pl.BlockSpec((bn, 64), lambda i: (i, 0)),
            pl.BlockSpec((bn, 1), lambda i: (i, 0)),
            pl.BlockSpec((64, _H), lambda i: (0, 0)),
            pl.BlockSpec((64, _H), lambda i: (0, 0)),
            pl.BlockSpec((1, _H), lambda i: (0, 0)),
            pl.BlockSpec((_H, _H), lambda i: (0, 0)),
            pl.BlockSpec((1, _H), lambda i: (0, 0)),
            pl.BlockSpec((_NG, bn), lambda i: (0, i)),
            pl.BlockSpec((_H, 64), lambda i: (0, 0)),
            pl.BlockSpec((1, 64), lambda i: (0, 0)),
            pl.BlockSpec((64, _H), lambda i: (0, 0)),
            pl.BlockSpec((1, _H), lambda i: (0, 0)),
        ],
        out_specs=[pl.BlockSpec((_NG, _H), lambda i: (0, 0))] * 3,
        out_shape=outs,
    )(s1a, s1b, aa, ab, deg, g2w[:64], g2w[64:], g2b, saw, sab, onehot,
      l1w, l1b, l2wp, l2bp)
    return out[:, :1]


_NPAD = 10240
_CH = 80
_NW = 32
_EPW = _E // _NW
_NCHUNK = _EPW // _CH


_SEG = 2000
_NSEG = _EPW // _SEG
_GRP = _SEG // 16
_SCPARAMS = pltpu.CompilerParams(use_tc_tiling_on_sc=False,
                                 needs_layout_passes=False)


def _rmw_min(tbl_v, idx16, val16, act16):
    t0 = plsc.load_gather(tbl_v, [idx16])
    m0 = act16 & (val16 < t0)
    cnt0 = jnp.sum(m0.astype(jnp.int32))

    def cond(cnt):
        return cnt > 0

    def body(cnt):
        t = plsc.load_gather(tbl_v, [idx16])
        m = act16 & (val16 < t)
        plsc.store_scatter(tbl_v, [idx16], val16, mask=m)
        t2 = plsc.load_gather(tbl_v, [idx16])
        m2 = act16 & (val16 < t2)
        return jnp.sum(m2.astype(jnp.int32))

    lax.while_loop(cond, body, cnt0)


def _tile_combine(tbl_v, sh_tbl, rbuf, racc, out_h, c, s, kind):
    zsl = _NPAD // 16
    pltpu.sync_copy(tbl_v, sh_tbl.at[s])
    plsc.subcore_barrier()
    pltpu.sync_copy(sh_tbl.at[0, pl.ds(s * zsl, zsl)], racc)
    for r in range(1, 16):
        pltpu.sync_copy(sh_tbl.at[r, pl.ds(s * zsl, zsl)], rbuf)

        def red(q, _, r=r):
            a = racc[pl.ds(q * 16, 16)]
            b = rbuf[pl.ds(q * 16, 16)]
            racc[pl.ds(q * 16, 16)] = (jnp.minimum(a, b) if kind == "min"
                                       else a + b)
            return 0

        lax.fori_loop(0, zsl // 16, red, 0)
    pltpu.sync_copy(racc, out_h.at[pl.ds(c * _NPAD + s * zsl, zsl)])


def _sc_geo1(posx, posy, posz, srcf, dstf, bigf):
    mesh = plsc.VectorSubcoreMesh(core_axis_name="c", subcore_axis_name="s")
    out_type = [jax.ShapeDtypeStruct((_E,), jnp.float32),
                jax.ShapeDtypeStruct((2 * _NPAD,), jnp.float32),
                jax.ShapeDtypeStruct((2 * _NPAD,), jnp.float32)]
    scratch = [pltpu.VMEM((_NPAD,), jnp.float32)] * 3 \
        + [pltpu.VMEM((_SEG,), jnp.int32)] * 2 \
        + [pltpu.VMEM((_SEG,), jnp.float32)] \
        + [pltpu.VMEM((_NPAD,), jnp.float32)] * 2 \
        + [pltpu.VMEM_SHARED((16, _NPAD), jnp.float32)] \
        + [pltpu.VMEM((_NPAD // 16,), jnp.float32)] * 2

    @functools.partial(pl.kernel, mesh=mesh, out_type=out_type,
                       scratch_types=scratch, compiler_params=_SCPARAMS)
    def k(posx_h, posy_h, posz_h, src_h, dst_h, big_h, d2_o, min1_o, deg_o,
          px, py, pz, sb, db, d2b, tmin, tdeg, sh_tbl, rbuf, racc):
        c = lax.axis_index("c")
        s = lax.axis_index("s")
        w = s * 2 + c
        pltpu.sync_copy(posx_h, px)
        pltpu.sync_copy(posy_h, py)
        pltpu.sync_copy(posz_h, pz)
        pltpu.sync_copy(big_h, tmin)
        ones16 = jnp.full((16,), 1.0, jnp.float32)

        def zero(q, _):
            tdeg[pl.ds(q * 16, 16)] = jnp.zeros((16,), jnp.float32)
            return 0

        lax.fori_loop(0, _NPAD // 16, zero, 0)
        true16 = jnp.full((16,), True)
        for seg in range(_NSEG):
            base = w * _EPW + seg * _SEG
            pltpu.sync_copy(src_h.at[pl.ds(base, _SEG)], sb)
            pltpu.sync_copy(dst_h.at[pl.ds(base, _SEG)], db)

            def grp(i, _):
                s16 = sb[pl.ds(i * 16, 16)]
                t16 = db[pl.ds(i * 16, 16)]
                dx = (plsc.load_gather(px, [t16])
                      - plsc.load_gather(px, [s16]))
                dy = (plsc.load_gather(py, [t16])
                      - plsc.load_gather(py, [s16]))
                dz = (plsc.load_gather(pz, [t16])
                      - plsc.load_gather(pz, [s16]))
                d2 = dx * dx + dy * dy + dz * dz
                d2b[pl.ds(i * 16, 16)] = d2
                _rmw_min(tmin, s16, d2, true16)
                plsc.addupdate_scatter(tdeg, [t16], ones16)
                return 0

            lax.fori_loop(0, _GRP, grp, 0)
            pltpu.sync_copy(d2b, d2_o.at[pl.ds(base, _SEG)])
        _tile_combine(tmin, sh_tbl, rbuf, racc, min1_o, c, s, "min")
        plsc.subcore_barrier()
        _tile_combine(tdeg, sh_tbl, rbuf, racc, deg_o, c, s, "sum")

    return k(posx, posy, posz, srcf, dstf, bigf)


def _sc_rmw_pass(mode, d2e, srcf, dstf, init_tbl, glbs):
    tdt = jnp.int32 if mode in ("nn1", "nn2") else jnp.float32
    gdts = {"nn1": [jnp.float32], "min2": [jnp.int32],
            "nn2": [jnp.int32, jnp.float32]}[mode]
    nglb = len(glbs)
    mesh = plsc.VectorSubcoreMesh(core_axis_name="c", subcore_axis_name="s")
    out_type = jax.ShapeDtypeStruct((2 * _NPAD,), tdt)
    scratch = ([pltpu.VMEM((_SEG,), jnp.int32)] * 2
               + [pltpu.VMEM((_SEG,), jnp.float32)]
               + [pltpu.VMEM((_NPAD,), g) for g in gdts]
               + [pltpu.VMEM((_NPAD,), tdt)]
               + [pltpu.VMEM_SHARED((16, _NPAD), tdt)]
               + [pltpu.VMEM((_NPAD // 16,), tdt)] * 2)

    @functools.partial(pl.kernel, mesh=mesh, out_type=out_type,
                       scratch_types=scratch, compiler_params=_SCPARAMS)
    def k(*refs):
        d2_h, src_h, dst_h, init_h = refs[:4]
        glb_h = refs[4:4 + nglb]
        out_o = refs[4 + nglb]
        sb, db, d2b = refs[5 + nglb:8 + nglb]
        glb_v = refs[8 + nglb:8 + 2 * nglb]
        tloc = refs[8 + 2 * nglb]
        sh_tbl, rbuf, racc = refs[9 + 2 * nglb:12 + 2 * nglb]
        c = lax.axis_index("c")
        s = lax.axis_index("s")
        w = s * 2 + c
        for gh, gv in zip(glb_h, glb_v):
            pltpu.sync_copy(gh, gv)
        pltpu.sync_copy(init_h, tloc)
        for seg in range(_NSEG):
            base = w * _EPW + seg * _SEG
            pltpu.sync_copy(src_h.at[pl.ds(base, _SEG)], sb)
            pltpu.sync_copy(dst_h.at[pl.ds(base, _SEG)], db)
            pltpu.sync_copy(d2_h.at[pl.ds(base, _SEG)], d2b)

            def grp(i, _):
                s16 = sb[pl.ds(i * 16, 16)]
                t16 = db[pl.ds(i * 16, 16)]
                d216 = d2b[pl.ds(i * 16, 16)]
                if mode == "nn1":
                    act = d216 == plsc.load_gather(glb_v[0], [s16])
                    val = t16
                elif mode == "min2":
                    act = plsc.load_gather(glb_v[0], [s16]) != t16
                    val = d216
                else:
                    act = ((plsc.load_gather(glb_v[0], [s16]) != t16)
                           & (d216 == plsc.load_gather(glb_v[1], [s16])))
                    val = t16
                _rmw_min(tloc, s16, val, act)
                return 0

            lax.fori_loop(0, _GRP, grp, 0)
        _tile_combine(tloc, sh_tbl, rbuf, racc, out_o, c, s, "min")

    return k(d2e, srcf, dstf, init_tbl, *glbs)


def _sc_geo5(posx, posy, posz, srcf, dstf, nn1, nn2, min1, min2):
    mesh = plsc.VectorSubcoreMesh(core_axis_name="c", subcore_axis_name="s")
    out_type = jax.ShapeDtypeStruct((_EPAD, 16), jnp.float32)
    scratch = ([pltpu.VMEM((_NPAD,), jnp.float32)] * 3
               + [pltpu.VMEM((_NPAD,), jnp.int32)] * 2
               + [pltpu.VMEM((_NPAD,), jnp.float32)] * 2
               + [pltpu.VMEM((_SEG,), jnp.int32)] * 2
               + [pltpu.VMEM((_SEG, 16), jnp.float32)])

    @functools.partial(pl.kernel, mesh=mesh, out_type=out_type,
                       scratch_types=scratch, compiler_params=_SCPARAMS)
    def k(posx_h, posy_h, posz_h, src_h, dst_h, nn1_h, nn2_h, m1_h, m2_h,
          vd_o, px, py, pz, n1v, n2v, m1v, m2v, sb, db, stag):
        c = lax.axis_index("c")
        s = lax.axis_index("s")
        w = s * 2 + c
        pltpu.sync_copy(posx_h, px)
        pltpu.sync_copy(posy_h, py)
        pltpu.sync_copy(posz_h, pz)
        pltpu.sync_copy(nn1_h, n1v)
        pltpu.sync_copy(nn2_h, n2v)
        pltpu.sync_copy(m1_h, m1v)
        pltpu.sync_copy(m2_h, m2v)
        lanes = lax.iota(jnp.int32, 16)
        for seg in range(_NSEG):
            base = w * _EPW + seg * _SEG
            pltpu.sync_copy(src_h.at[pl.ds(base, _SEG)], sb)
            pltpu.sync_copy(dst_h.at[pl.ds(base, _SEG)], db)

            def grp(i, _):
                s16 = sb[pl.ds(i * 16, 16)]
                t16 = db[pl.ds(i * 16, 16)]
                lg = plsc.load_gather
                n1s, n2s = lg(n1v, [s16]), lg(n2v, [s16])
                h1s, h2s = lg(m1v, [s16]) < _BIG, lg(m2v, [s16]) < _BIG
                n1d, n2d = lg(n1v, [t16]), lg(n2v, [t16])
                h1d, h2d = lg(m1v, [t16]) < _BIG, lg(m2v, [t16]) < _BIG
                fi = jnp.where(~h1s, s16,
                               jnp.where(n1s != t16, n1s,
                                         jnp.where(h2s, n2s, s16)))
                fj = jnp.where(~h1d, t16,
                               jnp.where(n1d != s16, n1d,
                                         jnp.where(h2d, n2d, t16)))
                pxs, pys, pzs = lg(px, [s16]), lg(py, [s16]), lg(pz, [s16])
                pxt, pyt, pzt = lg(px, [t16]), lg(py, [t16]), lg(pz, [t16])
                v1 = (pxs - lg(px, [fi]), pys - lg(py, [fi]),
                      pzs - lg(pz, [fi]))
                v2 = (pxt - pxs, pyt - pys, pzt - pzs)
                v3 = (pxt - lg(px, [fj]), pyt - lg(py, [fj]),
                      pzt - lg(pz, [fj]))
                d2 = v2[0] * v2[0] + v2[1] * v2[1] + v2[2] * v2[2]
                rows16 = lanes + i * 16
                cols = v1 + v2 + v3 + (d2,)
                for colid, cv in enumerate(cols):
                    plsc.store_scatter(
                        stag, [rows16, jnp.full((16,), colid, jnp.int32)], cv)
                return 0

            lax.fori_loop(0, _GRP, grp, 0)
            pltpu.sync_copy(stag, vd_o.at[pl.ds(base, _SEG)])

    return k(posx, posy, posz, srcf, dstf, nn1, nn2, min1, min2)


def _sc_agg_call(tbl, gfs, src3, dst3, zeros):
    ngf = len(gfs)
    ntbl = 0 if tbl is None else 1
    nacc = ntbl + ngf
    assert nacc <= 2
    R = 5 if nacc == 1 else 1
    mesh = plsc.VectorSubcoreMesh(core_axis_name="c", subcore_axis_name="s")
    out_type = [jax.ShapeDtypeStruct((2 * _NPAD, 64), jnp.float32)] * nacc
    scratch = ([pltpu.VMEM((_NCHUNK, _CH), jnp.int32),
                pltpu.VMEM((_NCHUNK, _CH), jnp.int32)]
               + [pltpu.VMEM((R, _CH, 64), jnp.float32)] * nacc
               + [pltpu.VMEM_SHARED((_NPAD, 64), jnp.float32)] * nacc
               + [pltpu.SemaphoreType.DMA] * (R * nacc))

    @functools.partial(pl.kernel, mesh=mesh, out_type=out_type,
                       scratch_types=scratch, compiler_params=_SCPARAMS)
    def k(*refs):
        tbl_h = refs[0] if ntbl else None
        rest = refs[ntbl:]
        gf_h = rest[:ngf]
        s3, d3, zer_h = rest[ngf:ngf + 3]
        outs = rest[ngf + 3:ngf + 3 + nacc]
        sidx, didx = rest[ngf + 3 + nacc:ngf + 5 + nacc]
        rings = rest[ngf + 5 + nacc:ngf + 5 + 2 * nacc]
        accs = rest[ngf + 5 + 2 * nacc:ngf + 5 + 3 * nacc]
        sems = rest[ngf + 5 + 3 * nacc:]
        c = lax.axis_index("c")
        s = lax.axis_index("s")
        w = s * 2 + c
        zslice = _NPAD // 16
        for acc in accs:
            pltpu.sync_copy(zer_h.at[pl.ds(s * zslice, zslice)],
                            acc.at[pl.ds(s * zslice, zslice)])
        pltpu.sync_copy(s3.at[w], sidx)
        pltpu.sync_copy(d3.at[w], didx)
        plsc.subcore_barrier()

        def load_desc(a, r, j):
            if a < ntbl:
                src = tbl_h.at[sidx.at[j]]
            else:
                src = gf_h[a - ntbl].at[w * _NCHUNK + j]
            return pltpu.make_async_copy(src, rings[a].at[r],
                                         sems[a * R + r])

        for r in range(R):
            for a in range(nacc):
                load_desc(a, r, r).start()

        def round_(j0, carry):
            for r in range(R):
                j = j0 * R + r
                for a in range(nacc):
                    load_desc(a, r, j).wait()
                di = didx.at[j]
                for a in range(nacc):
                    pltpu.sync_copy(rings[a].at[r], accs[a].at[di], add=True)
                jn = j + R

                @pl.when(jn < _NCHUNK)
                def _(r=r, jn=jn):
                    for a in range(nacc):
                        load_desc(a, r, jn).start()

            return carry

        lax.fori_loop(0, _NCHUNK // R, round_, 0)
        plsc.subcore_barrier()
        for acc, out in zip(accs, outs):
            pltpu.sync_copy(acc.at[pl.ds(s * zslice, zslice)],
                            out.at[pl.ds(c * _NPAD + s * zslice, zslice)])

    args = ([] if tbl is None else [tbl]) + list(gfs) + [src3, dst3, zeros]
    res = k(*args)
    res = res if isinstance(res, (list, tuple)) else [res]
    return [r[:_NPAD] + r[_NPAD:] for r in res]


def kernel(x, edge_index, batch, pos, c1_lin_w, c1_lin_b, c1_g1_w, c1_g1_b,
           c1_g2_w, c1_g2_b, c2_lin_w, c2_lin_b, c2_g1_w, c2_g1_b, c2_g2_w,
           c2_g2_b, sa_w, sa_b, l1_w, l1_b, l2_w, l2_b):
    src = edge_index[0].astype(jnp.int32)
    dst = edge_index[1].astype(jnp.int32)

    posp = jnp.pad(pos, ((0, _NPAD - _N), (0, 0)))
    posx, posy, posz = posp[:, 0], posp[:, 1], posp[:, 2]
    bigf = jnp.full((_NPAD,), _BIG, jnp.float32)
    sentn = jnp.full((_NPAD,), _N, jnp.int32)
    d2e, min1p, degp = _sc_geo1(posx, posy, posz, src, dst, bigf)
    min1 = jnp.minimum(min1p[:_NPAD], min1p[_NPAD:])
    deg = (degp[:_NPAD] + degp[_NPAD:])[:_N, None]
    nn1p = _sc_rmw_pass("nn1", d2e, src, dst, sentn, [min1])
    nn1 = jnp.minimum(nn1p[:_NPAD], nn1p[_NPAD:])
    min2p = _sc_rmw_pass("min2", d2e, src, dst, bigf, [nn1])
    min2 = jnp.minimum(min2p[:_NPAD], min2p[_NPAD:])
    nn2p = _sc_rmw_pass("nn2", d2e, src, dst, sentn, [nn1, min2])
    nn2 = jnp.minimum(nn2p[:_NPAD], nn2p[_NPAD:])
    vd = _sc_geo5(posx, posy, posz, src, dst, nn1, nn2, min1, min2)

    gf1a, gf1b, gf2a, gf2b = _trig_gf(
        vd, c1_g1_w, c1_g1_b.reshape(1, _H), c2_g1_w, c2_g1_b.reshape(1, _H))

    src3 = src.reshape(_NW, _NCHUNK, _CH)
    dst3 = dst.reshape(_NW, _NCHUNK, _CH)
    zeros = jnp.zeros((_NPAD, 64), jnp.float32)
    rs = lambda g: g[:_E].reshape(_E // _CH, _CH, 64)

    hh1a, hh1b = _nodelin(x, c1_lin_w, c1_lin_b.reshape(1, _H))
    s1a, a1a = _sc_agg_call(hh1a, [rs(gf1a)], src3, dst3, zeros)
    s1b, a2a = _sc_agg_call(hh1b, [rs(gf2a)], src3, dst3, zeros)
    a1b, a2b = _sc_agg_call(None, [rs(gf1b), rs(gf2b)], src3, dst3, zeros)
    s1a, s1b, a1a, a1b = s1a[:_N], s1b[:_N], a1a[:_N], a1b[:_N]

    hh2a, hh2b = _combine_lin(s1a, s1b, a1a, a1b, deg, c1_g2_w,
                              c1_g2_b.reshape(1, _H), c2_lin_w,
                              c2_lin_b.reshape(1, _H))
    (s2a,) = _sc_agg_call(hh2a, [], src3, dst3, zeros)
    (s2b,) = _sc_agg_call(hh2b, [], src3, dst3, zeros)
    s2a, s2b, a2a, a2b = s2a[:_N], s2b[:_N], a2a[:_N], a2b[:_N]

    onehot = (batch[None, :].astype(jnp.int32)
              == jnp.arange(_NG, dtype=jnp.int32)[:, None]).astype(jnp.float32)
    l2wp = jnp.pad(l2_w, ((0, 0), (0, _H - 1)))
    l2bp = jnp.pad(l2_b, (0, _H - 1)).reshape(1, _H)
    return _final_head(s2a, s2b, a2a, a2b, deg, c2_g2_w,
                       c2_g2_b.reshape(1, _H), sa_w, sa_b.reshape(1, _H),
                       onehot, l1_w, l1_b.reshape(1, 64), l2wp, l2bp)


# speedup vs baseline: 12.1542x; 1.0494x over previous
import functools
import jax
import jax.numpy as jnp
from jax import lax
from jax.experimental import pallas as pl
from jax.experimental.pallas import tpu as pltpu
from jax.experimental.pallas import tpu_sc as plsc

_N = 10000
_E = 320000
_EPAD = 327680
_H = 128
_NG = 64
_EPS = 1e-8
_BIG = 1e30


def _trig_gf_body(vd_ref, w1_ref, b1_ref, w2_ref, b2_ref, o1a, o1b, o2a, o2b):
    vd = vd_ref[...]
    v1x, v1y, v1z = vd[:, 0:1], vd[:, 1:2], vd[:, 2:3]
    v2x, v2y, v2z = vd[:, 3:4], vd[:, 4:5], vd[:, 5:6]
    v3x, v3y, v3z = vd[:, 6:7], vd[:, 7:8], vd[:, 8:9]
    d2 = vd[:, 9:10]
    d = jnp.sqrt(d2)
    ct = jnp.clip(v2z / (d + _EPS), -1.0 + _EPS, 1.0 - _EPS)
    theta = jnp.arctan2(jnp.sqrt(jnp.maximum(1.0 - ct * ct, 0.0)), ct)
    phi = jnp.arctan2(v2y, v2x)
    n1x = v1y * v2z - v1z * v2y
    n1y = v1z * v2x - v1x * v2z
    n1z = v1x * v2y - v1y * v2x
    n2x = v2y * v3z - v2z * v3y
    n2y = v2z * v3x - v2x * v3z
    n2z = v2x * v3y - v2y * v3x
    dotn = n1x * n2x + n1y * n2y + n1z * n2z
    nn1 = jnp.sqrt(n1x * n1x + n1y * n1y + n1z * n1z) + _EPS
    nn2 = jnp.sqrt(n2x * n2x + n2y * n2y + n2z * n2z) + _EPS
    ctau = jnp.clip(dotn / (nn1 * nn2), -1.0 + _EPS, 1.0 - _EPS)
    tau = jnp.arctan2(jnp.sqrt(jnp.maximum(1.0 - ctau * ctau, 0.0)), ctau)

    def gf(w_ref, b_ref):
        w = w_ref[...]
        return jnp.maximum(d * w[0:1, :] + theta * w[1:2, :] + phi * w[2:3, :]
                           + tau * w[3:4, :] + b_ref[...], 0.0)

    g1 = gf(w1_ref, b1_ref)
    g2 = gf(w2_ref, b2_ref)
    o1a[...] = g1[:, :64]
    o1b[...] = g1[:, 64:]
    o2a[...] = g2[:, :64]
    o2b[...] = g2[:, 64:]


def _trig_gf(vd, w1, b1, w2, b2):
    be = 4096
    grid = (_EPAD // be,)
    half = jax.ShapeDtypeStruct((_EPAD, 64), jnp.float32)
    return pl.pallas_call(
        _trig_gf_body,
        grid=grid,
        in_specs=[
            pl.BlockSpec((be, 16), lambda i: (i, 0)),
            pl.BlockSpec((4, _H), lambda i: (0, 0)),
            pl.BlockSpec((1, _H), lambda i: (0, 0)),
            pl.BlockSpec((4, _H), lambda i: (0, 0)),
            pl.BlockSpec((1, _H), lambda i: (0, 0)),
        ],
        out_specs=[pl.BlockSpec((be, 64), lambda i: (i, 0))] * 4,
        out_shape=[half, half, half, half],
    )(vd, w1, b1, w2, b2)


def _nodelin_body(h_ref, w_ref, b_ref, oa, ob):
    hh = jnp.dot(h_ref[...], w_ref[...],
                 preferred_element_type=jnp.float32) + b_ref[...]
    oa[...] = hh[:, :64]
    ob[...] = hh[:, 64:]


def _nodelin(h, w, b):
    bn = 2000
    half = jax.ShapeDtypeStruct((_N, 64), jnp.float32)
    return pl.pallas_call(
        _nodelin_body,
        grid=(_N // bn,),
        in_specs=[
            pl.BlockSpec((bn, _H), lambda i: (i, 0)),
            pl.BlockSpec((_H, _H), lambda i: (0, 0)),
            pl.BlockSpec((1, _H), lambda i: (0, 0)),
        ],
        out_specs=[pl.BlockSpec((bn, 64), lambda i: (i, 0))] * 2,
        out_shape=[half, half],
    )(h, w, b)


def _combine_body(s1a, s1b, aa, ab, deg_ref, g2wa, g2wb, g2b, lw, lb, oa, ob):
    acont = (jnp.dot(aa[...], g2wa[...], preferred_element_type=jnp.float32)
             + jnp.dot(ab[...], g2wb[...], preferred_element_type=jnp.float32))
    s1 = jnp.concatenate([s1a[...], s1b[...]], axis=1)
    h = jnp.maximum(s1 + acont + deg_ref[...] * g2b[...], 0.0)
    hh = jnp.dot(h, lw[...], preferred_element_type=jnp.float32) + lb[...]
    oa[...] = hh[:, :64]
    ob[...] = hh[:, 64:]


def _combine_lin(s1a, s1b, aa, ab, deg, g2w, g2b, lw, lb):
    bn = 2000
    half = jax.ShapeDtypeStruct((_N, 64), jnp.float32)
    return pl.pallas_call(
        _combine_body,
        grid=(_N // bn,),
        in_specs=[
            pl.BlockSpec((bn, 64), lambda i: (i, 0)),
            pl.BlockSpec((bn, 64), lambda i: (i, 0)),
            pl.BlockSpec((bn, 64), lambda i: (i, 0)),
            pl.BlockSpec((bn, 64), lambda i: (i, 0)),
            pl.BlockSpec((bn, 1), lambda i: (i, 0)),
            pl.BlockSpec((64, _H), lambda i: (0, 0)),
            pl.BlockSpec((64, _H), lambda i: (0, 0)),
            pl.BlockSpec((1, _H), lambda i: (0, 0)),
            pl.BlockSpec((_H, _H), lambda i: (0, 0)),
            pl.BlockSpec((1, _H), lambda i: (0, 0)),
        ],
        out_specs=[pl.BlockSpec((bn, 64), lambda i: (i, 0))] * 2,
        out_shape=[half, half],
    )(s1a, s1b, aa, ab, deg, g2w[:64], g2w[64:], g2b, lw, lb)


def _final_body(s1a, s1b, aa, ab, deg_ref, g2wa, g2wb, g2b, saw, sab,
                oh_ref, l1w, l1b, l2w, l2b, sums_ref, cnt_ref, out_ref):
    i = pl.program_id(0)
    nsteps = pl.num_programs(0)
    acont = (jnp.dot(aa[...], g2wa[...], preferred_element_type=jnp.float32)
             + jnp.dot(ab[...], g2wb[...], preferred_element_type=jnp.float32))
    s1 = jnp.concatenate([s1a[...], s1b[...]], axis=1)
    h2 = jnp.maximum(s1 + acont + deg_ref[...] * g2b[...], 0.0)
    h3 = jnp.maximum(jnp.dot(h2, saw[...],
                             preferred_element_type=jnp.float32) + sab[...], 0.0)
    oh = oh_ref[...]
    psum = jnp.dot(oh, h3, preferred_element_type=jnp.float32)
    ones = jnp.ones((oh.shape[1], _H), jnp.float32)
    pcnt = jnp.dot(oh, ones, preferred_element_type=jnp.float32)

    @pl.when(i == 0)
    def _():
        sums_ref[...] = psum
        cnt_ref[...] = pcnt

    @pl.when(i > 0)
    def _():
        sums_ref[...] += psum
        cnt_ref[...] += pcnt

    @pl.when(i == nsteps - 1)
    def _():
        cnt = cnt_ref[...]
        sums = sums_ref[...]
        pooled = jnp.where(cnt > 0, sums / jnp.maximum(cnt, 1.0), 0.0)
        hf = jnp.maximum(jnp.dot(pooled, l1w[...],
                                 preferred_element_type=jnp.float32) + l1b[...], 0.0)
        out_ref[...] = jnp.dot(hf, l2w[...],
                               preferred_element_type=jnp.float32) + l2b[...]


def _final_head(s1a, s1b, aa, ab, deg, g2w, g2b, saw, sab, onehot,
                l1w, l1b, l2wp, l2bp):
    bn = 2048
    npad = 10240
    pad = lambda t: jnp.pad(t, ((0, npad - _N), (0, 0)))
    s1a, s1b, aa, ab, deg = map(pad, (s1a, s1b, aa, ab, deg))
    onehot = jnp.pad(onehot, ((0, 0), (0, npad - _N)))
    outs = [jax.ShapeDtypeStruct((_NG, _H), jnp.float32),
            jax.ShapeDtypeStruct((_NG, _H), jnp.float32),
            jax.ShapeDtypeStruct((_NG, _H), jnp.float32)]
    sums, cnt, out = pl.pallas_call(
        _final_body,
        grid=(npad // bn,),
        in_specs=[
            pl.BlockSpec((bn, 64), lambda i: (i, 0)),
            pl.BlockSpec((bn, 64), lambda i: (i, 0)),
            pl.BlockSpec((bn, 64), lambda i: (i, 0)),
            pl.BlockSpec((bn, 64), lambda i: (i, 0)),
            pl.BlockSpec((bn, 1), lambda i: (i, 0)),
            pl.BlockSpec((64, _H), lambda i: (0, 0)),
            pl.BlockSpec((64, _H), lambda i: (0, 0)),
            pl.BlockSpec((1, _H), lambda i: (0, 0)),
            pl.BlockSpec((_H, _H), lambda i: (0, 0)),
            pl.BlockSpec((1, _H), lambda i: (0, 0)),
            pl.BlockSpec((_NG, bn), lambda i: (0, i)),
            pl.BlockSpec((_H, 64), lambda i: (0, 0)),
            pl.BlockSpec((1, 64), lambda i: (0, 0)),
            pl.BlockSpec((64, _H), lambda i: (0, 0)),
            pl.BlockSpec((1, _H), lambda i: (0, 0)),
        ],
        out_specs=[pl.BlockSpec((_NG, _H), lambda i: (0, 0))] * 3,
        out_shape=outs,
    )(s1a, s1b, aa, ab, deg, g2w[:64], g2w[64:], g2b, saw, sab, onehot,
      l1w, l1b, l2wp, l2bp)
    return out[:, :1]


_NPAD = 10240
_CH = 80
_NW = 32
_EPW = _E // _NW
_NCHUNK = _EPW // _CH


_SEG = 2000
_NSEG = _EPW // _SEG
_GRP = _SEG // 16
_SCPARAMS = pltpu.CompilerParams(use_tc_tiling_on_sc=False,
                                 needs_layout_passes=False)


def _rmw_min(tbl_v, idx16, val16, act16):
    t0 = plsc.load_gather(tbl_v, [idx16])
    m0 = act16 & (val16 < t0)
    cnt0 = jnp.sum(m0.astype(jnp.int32))

    def cond(cnt):
        return cnt > 0

    def body(cnt):
        t = plsc.load_gather(tbl_v, [idx16])
        m = act16 & (val16 < t)
        plsc.store_scatter(tbl_v, [idx16], val16, mask=m)
        t2 = plsc.load_gather(tbl_v, [idx16])
        m2 = act16 & (val16 < t2)
        return jnp.sum(m2.astype(jnp.int32))

    lax.while_loop(cond, body, cnt0)


def _tile_combine(tbl_v, sh_tbl, rbuf, racc, out_h, c, s, kind):
    zsl = _NPAD // 16
    pltpu.sync_copy(tbl_v, sh_tbl.at[s])
    plsc.subcore_barrier()
    pltpu.sync_copy(sh_tbl.at[0, pl.ds(s * zsl, zsl)], racc)
    for r in range(1, 16):
        pltpu.sync_copy(sh_tbl.at[r, pl.ds(s * zsl, zsl)], rbuf)

        def red(q, _, r=r):
            a = racc[pl.ds(q * 16, 16)]
            b = rbuf[pl.ds(q * 16, 16)]
            racc[pl.ds(q * 16, 16)] = (jnp.minimum(a, b) if kind == "min"
                                       else a + b)
            return 0

        lax.fori_loop(0, zsl // 16, red, 0)
    pltpu.sync_copy(racc, out_h.at[pl.ds(c * _NPAD + s * zsl, zsl)])


def _sc_geo1(posx, posy, posz, srcf, dstf, bigf):
    mesh = plsc.VectorSubcoreMesh(core_axis_name="c", subcore_axis_name="s")
    out_type = [jax.ShapeDtypeStruct((_E,), jnp.float32),
                jax.ShapeDtypeStruct((2 * _NPAD,), jnp.float32),
                jax.ShapeDtypeStruct((2 * _NPAD,), jnp.float32)]
    scratch = [pltpu.VMEM((_NPAD,), jnp.float32)] * 3 \
        + [pltpu.VMEM((_SEG,), jnp.int32)] * 2 \
        + [pltpu.VMEM((_SEG,), jnp.float32)] \
        + [pltpu.VMEM((_NPAD,), jnp.float32)] * 2 \
        + [pltpu.VMEM_SHARED((16, _NPAD), jnp.float32)] \
        + [pltpu.VMEM((_NPAD // 16,), jnp.float32)] * 2

    @functools.partial(pl.kernel, mesh=mesh, out_type=out_type,
                       scratch_types=scratch, compiler_params=_SCPARAMS)
    def k(posx_h, posy_h, posz_h, src_h, dst_h, big_h, d2_o, min1_o, deg_o,
          px, py, pz, sb, db, d2b, tmin, tdeg, sh_tbl, rbuf, racc):
        c = lax.axis_index("c")
        s = lax.axis_index("s")
        w = s * 2 + c
        pltpu.sync_copy(posx_h, px)
        pltpu.sync_copy(posy_h, py)
        pltpu.sync_copy(posz_h, pz)
        pltpu.sync_copy(big_h, tmin)
        ones16 = jnp.full((16,), 1.0, jnp.float32)

        def zero(q, _):
            tdeg[pl.ds(q * 16, 16)] = jnp.zeros((16,), jnp.float32)
            return 0

        lax.fori_loop(0, _NPAD // 16, zero, 0)
        true16 = jnp.full((16,), True)
        for seg in range(_NSEG):
            base = w * _EPW + seg * _SEG
            pltpu.sync_copy(src_h.at[pl.ds(base, _SEG)], sb)
            pltpu.sync_copy(dst_h.at[pl.ds(base, _SEG)], db)

            def grp(i, _):
                s16 = sb[pl.ds(i * 16, 16)]
                t16 = db[pl.ds(i * 16, 16)]
                dx = (plsc.load_gather(px, [t16])
                      - plsc.load_gather(px, [s16]))
                dy = (plsc.load_gather(py, [t16])
                      - plsc.load_gather(py, [s16]))
                dz = (plsc.load_gather(pz, [t16])
                      - plsc.load_gather(pz, [s16]))
                d2 = dx * dx + dy * dy + dz * dz
                d2b[pl.ds(i * 16, 16)] = d2
                _rmw_min(tmin, s16, d2, true16)
                plsc.addupdate_scatter(tdeg, [t16], ones16)
                return 0

            lax.fori_loop(0, _GRP, grp, 0)
            pltpu.sync_copy(d2b, d2_o.at[pl.ds(base, _SEG)])
        _tile_combine(tmin, sh_tbl, rbuf, racc, min1_o, c, s, "min")
        plsc.subcore_barrier()
        _tile_combine(tdeg, sh_tbl, rbuf, racc, deg_o, c, s, "sum")

    return k(posx, posy, posz, srcf, dstf, bigf)


def _sc_rmw_pass(mode, d2e, srcf, dstf, init_tbl, glbs):
    tdt = jnp.int32 if mode in ("nn1", "nn2") else jnp.float32
    gdts = {"nn1": [jnp.float32], "min2": [jnp.int32],
            "nn2": [jnp.int32, jnp.float32]}[mode]
    nglb = len(glbs)
    mesh = plsc.VectorSubcoreMesh(core_axis_name="c", subcore_axis_name="s")
    out_type = jax.ShapeDtypeStruct((2 * _NPAD,), tdt)
    scratch = ([pltpu.VMEM((_SEG,), jnp.int32)] * 2
               + [pltpu.VMEM((_SEG,), jnp.float32)]
               + [pltpu.VMEM((_NPAD,), g) for g in gdts]
               + [pltpu.VMEM((_NPAD,), tdt)]
               + [pltpu.VMEM_SHARED((16, _NPAD), tdt)]
               + [pltpu.VMEM((_NPAD // 16,), tdt)] * 2)

    @functools.partial(pl.kernel, mesh=mesh, out_type=out_type,
                       scratch_types=scratch, compiler_params=_SCPARAMS)
    def k(*refs):
        d2_h, src_h, dst_h, init_h = refs[:4]
        glb_h = refs[4:4 + nglb]
        out_o = refs[4 + nglb]
        sb, db, d2b = refs[5 + nglb:8 + nglb]
        glb_v = refs[8 + nglb:8 + 2 * nglb]
        tloc = refs[8 + 2 * nglb]
        sh_tbl, rbuf, racc = refs[9 + 2 * nglb:12 + 2 * nglb]
        c = lax.axis_index("c")
        s = lax.axis_index("s")
        w = s * 2 + c
        for gh, gv in zip(glb_h, glb_v):
            pltpu.sync_copy(gh, gv)
        pltpu.sync_copy(init_h, tloc)
        for seg in range(_NSEG):
            base = w * _EPW + seg * _SEG
            pltpu.sync_copy(src_h.at[pl.ds(base, _SEG)], sb)
            pltpu.sync_copy(dst_h.at[pl.ds(base, _SEG)], db)
            pltpu.sync_copy(d2_h.at[pl.ds(base, _SEG)], d2b)

            def grp(i, _):
                s16 = sb[pl.ds(i * 16, 16)]
                t16 = db[pl.ds(i * 16, 16)]
                d216 = d2b[pl.ds(i * 16, 16)]
                if mode == "nn1":
                    act = d216 == plsc.load_gather(glb_v[0], [s16])
                    val = t16
                elif mode == "min2":
                    act = plsc.load_gather(glb_v[0], [s16]) != t16
                    val = d216
                else:
                    act = ((plsc.load_gather(glb_v[0], [s16]) != t16)
                           & (d216 == plsc.load_gather(glb_v[1], [s16])))
                    val = t16
                _rmw_min(tloc, s16, val, act)
                return 0

            lax.fori_loop(0, _GRP, grp, 0)
        _tile_combine(tloc, sh_tbl, rbuf, racc, out_o, c, s, "min")

    return k(d2e, srcf, dstf, init_tbl, *glbs)


def _sc_geo5(posx, posy, posz, srcf, dstf, nn1, nn2, min1, min2):
    mesh = plsc.VectorSubcoreMesh(core_axis_name="c", subcore_axis_name="s")
    out_type = jax.ShapeDtypeStruct((_EPAD, 16), jnp.float32)
    scratch = ([pltpu.VMEM((_NPAD,), jnp.float32)] * 3
               + [pltpu.VMEM((_NPAD,), jnp.int32)] * 2
               + [pltpu.VMEM((_NPAD,), jnp.float32)] * 2
               + [pltpu.VMEM((_SEG,), jnp.int32)] * 2
               + [pltpu.VMEM((_SEG, 16), jnp.float32)])

    @functools.partial(pl.kernel, mesh=mesh, out_type=out_type,
                       scratch_types=scratch, compiler_params=_SCPARAMS)
    def k(posx_h, posy_h, posz_h, src_h, dst_h, nn1_h, nn2_h, m1_h, m2_h,
          vd_o, px, py, pz, n1v, n2v, m1v, m2v, sb, db, stag):
        c = lax.axis_index("c")
        s = lax.axis_index("s")
        w = s * 2 + c
        pltpu.sync_copy(posx_h, px)
        pltpu.sync_copy(posy_h, py)
        pltpu.sync_copy(posz_h, pz)
        pltpu.sync_copy(nn1_h, n1v)
        pltpu.sync_copy(nn2_h, n2v)
        pltpu.sync_copy(m1_h, m1v)
        pltpu.sync_copy(m2_h, m2v)
        lanes = lax.iota(jnp.int32, 16)
        for seg in range(_NSEG):
            base = w * _EPW + seg * _SEG
            pltpu.sync_copy(src_h.at[pl.ds(base, _SEG)], sb)
            pltpu.sync_copy(dst_h.at[pl.ds(base, _SEG)], db)

            def grp(i, _):
                s16 = sb[pl.ds(i * 16, 16)]
                t16 = db[pl.ds(i * 16, 16)]
                lg = plsc.load_gather
                n1s, n2s = lg(n1v, [s16]), lg(n2v, [s16])
                h1s, h2s = lg(m1v, [s16]) < _BIG, lg(m2v, [s16]) < _BIG
                n1d, n2d = lg(n1v, [t16]), lg(n2v, [t16])
                h1d, h2d = lg(m1v, [t16]) < _BIG, lg(m2v, [t16]) < _BIG
                fi = jnp.where(~h1s, s16,
                               jnp.where(n1s != t16, n1s,
                                         jnp.where(h2s, n2s, s16)))
                fj = jnp.where(~h1d, t16,
                               jnp.where(n1d != s16, n1d,
                                         jnp.where(h2d, n2d, t16)))
                pxs, pys, pzs = lg(px, [s16]), lg(py, [s16]), lg(pz, [s16])
                pxt, pyt, pzt = lg(px, [t16]), lg(py, [t16]), lg(pz, [t16])
                v1 = (pxs - lg(px, [fi]), pys - lg(py, [fi]),
                      pzs - lg(pz, [fi]))
                v2 = (pxt - pxs, pyt - pys, pzt - pzs)
                v3 = (pxt - lg(px, [fj]), pyt - lg(py, [fj]),
                      pzt - lg(pz, [fj]))
                d2 = v2[0] * v2[0] + v2[1] * v2[1] + v2[2] * v2[2]
                rows16 = lanes + i * 16
                cols = v1 + v2 + v3 + (d2,)
                for colid, cv in enumerate(cols):
                    plsc.store_scatter(
                        stag, [rows16, jnp.full((16,), colid, jnp.int32)], cv)
                return 0

            lax.fori_loop(0, _GRP, grp, 0)
            pltpu.sync_copy(stag, vd_o.at[pl.ds(base, _SEG)])

    return k(posx, posy, posz, srcf, dstf, nn1, nn2, min1, min2)


def _sc_agg_call(tbl, gfs, src3, dst3, zeros):
    ngf = len(gfs)
    ntbl = 0 if tbl is None else 1
    nacc = ntbl + ngf
    assert nacc <= 2
    R = 5 if nacc == 1 else 1
    mesh = plsc.VectorSubcoreMesh(core_axis_name="c", subcore_axis_name="s")
    out_type = [jax.ShapeDtypeStruct((2 * _NPAD, 64), jnp.float32)] * nacc
    scratch = ([pltpu.VMEM((_NCHUNK, _CH), jnp.int32),
                pltpu.VMEM((_NCHUNK, _CH), jnp.int32)]
               + [pltpu.VMEM((R, _CH, 64), jnp.float32)] * nacc
               + [pltpu.VMEM_SHARED((_NPAD, 64), jnp.float32)] * nacc
               + [pltpu.SemaphoreType.DMA] * (R * nacc))

    @functools.partial(pl.kernel, mesh=mesh, out_type=out_type,
                       scratch_types=scratch, compiler_params=_SCPARAMS)
    def k(*refs):
        tbl_h = refs[0] if ntbl else None
        rest = refs[ntbl:]
        gf_h = rest[:ngf]
        s3, d3, zer_h = rest[ngf:ngf + 3]
        outs = rest[ngf + 3:ngf + 3 + nacc]
        sidx, didx = rest[ngf + 3 + nacc:ngf + 5 + nacc]
        rings = rest[ngf + 5 + nacc:ngf + 5 + 2 * nacc]
        accs = rest[ngf + 5 + 2 * nacc:ngf + 5 + 3 * nacc]
        sems = rest[ngf + 5 + 3 * nacc:]
        c = lax.axis_index("c")
        s = lax.axis_index("s")
        w = s * 2 + c
        zslice = _NPAD // 16
        for acc in accs:
            pltpu.sync_copy(zer_h.at[pl.ds(s * zslice, zslice)],
                            acc.at[pl.ds(s * zslice, zslice)])
        pltpu.sync_copy(s3.at[w], sidx)
        pltpu.sync_copy(d3.at[w], didx)
        plsc.subcore_barrier()

        def load_desc(a, r, j):
            if a < ntbl:
                src = tbl_h.at[sidx.at[j]]
            else:
                src = gf_h[a - ntbl].at[w * _NCHUNK + j]
            return pltpu.make_async_copy(src, rings[a].at[r],
                                         sems[a * R + r])

        for r in range(R):
            for a in range(nacc):
                load_desc(a, r, r).start()

        def round_(j0, carry):
            for r in range(R):
                j = j0 * R + r
                for a in range(nacc):
                    load_desc(a, r, j).wait()
                di = didx.at[j]
                for a in range(nacc):
                    pltpu.sync_copy(rings[a].at[r], accs[a].at[di], add=True)
                jn = j + R

                @pl.when(jn < _NCHUNK)
                def _(r=r, jn=jn):
                    for a in range(nacc):
                        load_desc(a, r, jn).start()

            return carry

        lax.fori_loop(0, _NCHUNK // R, round_, 0)
        plsc.subcore_barrier()
        for acc, out in zip(accs, outs):
            pltpu.sync_copy(acc.at[pl.ds(s * zslice, zslice)],
                            out.at[pl.ds(c * _NPAD + s * zslice, zslice)])

    args = ([] if tbl is None else [tbl]) + list(gfs) + [src3, dst3, zeros]
    res = k(*args)
    res = res if isinstance(res, (list, tuple)) else [res]
    return [r[:_NPAD] + r[_NPAD:] for r in res]


def kernel(x, edge_index, batch, pos, c1_lin_w, c1_lin_b, c1_g1_w, c1_g1_b,
           c1_g2_w, c1_g2_b, c2_lin_w, c2_lin_b, c2_g1_w, c2_g1_b, c2_g2_w,
           c2_g2_b, sa_w, sa_b, l1_w, l1_b, l2_w, l2_b):
    src = edge_index[0].astype(jnp.int32)
    dst = edge_index[1].astype(jnp.int32)

    posp = jnp.pad(pos, ((0, _NPAD - _N), (0, 0)))
    posx, posy, posz = posp[:, 0], posp[:, 1], posp[:, 2]
    bigf = jnp.full((_NPAD,), _BIG, jnp.float32)
    sentn = jnp.full((_NPAD,), _N, jnp.int32)
    d2e, min1p, degp = _sc_geo1(posx, posy, posz, src, dst, bigf)
    min1 = jnp.minimum(min1p[:_NPAD], min1p[_NPAD:])
    deg = (degp[:_NPAD] + degp[_NPAD:])[:_N, None]
    nn1p = _sc_rmw_pass("nn1", d2e, src, dst, sentn, [min1])
    nn1 = jnp.minimum(nn1p[:_NPAD], nn1p[_NPAD:])
    min2p = _sc_rmw_pass("min2", d2e, src, dst, bigf, [nn1])
    min2 = jnp.minimum(min2p[:_NPAD], min2p[_NPAD:])
    nn2p = _sc_rmw_pass("nn2", d2e, src, dst, sentn, [nn1, min2])
    nn2 = jnp.minimum(nn2p[:_NPAD], nn2p[_NPAD:])
    vd = _sc_geo5(posx, posy, posz, src, dst, nn1, nn2, min1, min2)

    gf1a, gf1b, gf2a, gf2b = _trig_gf(
        vd, c1_g1_w, c1_g1_b.reshape(1, _H), c2_g1_w, c2_g1_b.reshape(1, _H))

    src3 = src.reshape(_NW, _NCHUNK, _CH)
    dst3 = dst.reshape(_NW, _NCHUNK, _CH)
    zeros = jnp.zeros((_NPAD, 64), jnp.float32)
    rs = lambda g: g[:_E].reshape(_E // _CH, _CH, 64)

    hh1a, hh1b = _nodelin(x, c1_lin_w, c1_lin_b.reshape(1, _H))
    (s1a,) = _sc_agg_call(hh1a, [], src3, dst3, zeros)
    (s1b,) = _sc_agg_call(hh1b, [], src3, dst3, zeros)
    (a1a,) = _sc_agg_call(None, [rs(gf1a)], src3, dst3, zeros)
    (a2a,) = _sc_agg_call(None, [rs(gf2a)], src3, dst3, zeros)
    (a1b,) = _sc_agg_call(None, [rs(gf1b)], src3, dst3, zeros)
    (a2b,) = _sc_agg_call(None, [rs(gf2b)], src3, dst3, zeros)
    s1a, s1b, a1a, a1b = s1a[:_N], s1b[:_N], a1a[:_N], a1b[:_N]

    hh2a, hh2b = _combine_lin(s1a, s1b, a1a, a1b, deg, c1_g2_w,
                              c1_g2_b.reshape(1, _H), c2_lin_w,
                              c2_lin_b.reshape(1, _H))
    (s2a,) = _sc_agg_call(hh2a, [], src3, dst3, zeros)
    (s2b,) = _sc_agg_call(hh2b, [], src3, dst3, zeros)
    s2a, s2b, a2a, a2b = s2a[:_N], s2b[:_N], a2a[:_N], a2b[:_N]

    onehot = (batch[None, :].astype(jnp.int32)
              == jnp.arange(_NG, dtype=jnp.int32)[:, None]).astype(jnp.float32)
    l2wp = jnp.pad(l2_w, ((0, 0), (0, _H - 1)))
    l2bp = jnp.pad(l2_b, (0, _H - 1)).reshape(1, _H)
    return _final_head(s2a, s2b, a2a, a2b, deg, c2_g2_w,
                       c2_g2_b.reshape(1, _H), sa_w, sa_b.reshape(1, _H),
                       onehot, l1_w, l1_b.reshape(1, 64), l2wp, l2bp)


# speedup vs baseline: 12.1734x; 1.0016x over previous
import functools
import jax
import jax.numpy as jnp
from jax import lax
from jax.experimental import pallas as pl
from jax.experimental.pallas import tpu as pltpu
from jax.experimental.pallas import tpu_sc as plsc

_N = 10000
_E = 320000
_EPAD = 327680
_H = 128
_NG = 64
_EPS = 1e-8
_BIG = 1e30


def _trig_gf_body(vd_ref, w1_ref, b1_ref, w2_ref, b2_ref, o1a, o1b, o2a, o2b):
    vd = vd_ref[...]
    v1x, v1y, v1z = vd[:, 0:1], vd[:, 1:2], vd[:, 2:3]
    v2x, v2y, v2z = vd[:, 3:4], vd[:, 4:5], vd[:, 5:6]
    v3x, v3y, v3z = vd[:, 6:7], vd[:, 7:8], vd[:, 8:9]
    d2 = vd[:, 9:10]
    d = jnp.sqrt(d2)
    ct = jnp.clip(v2z / (d + _EPS), -1.0 + _EPS, 1.0 - _EPS)
    theta = jnp.arctan2(jnp.sqrt(jnp.maximum(1.0 - ct * ct, 0.0)), ct)
    phi = jnp.arctan2(v2y, v2x)
    n1x = v1y * v2z - v1z * v2y
    n1y = v1z * v2x - v1x * v2z
    n1z = v1x * v2y - v1y * v2x
    n2x = v2y * v3z - v2z * v3y
    n2y = v2z * v3x - v2x * v3z
    n2z = v2x * v3y - v2y * v3x
    dotn = n1x * n2x + n1y * n2y + n1z * n2z
    nn1 = jnp.sqrt(n1x * n1x + n1y * n1y + n1z * n1z) + _EPS
    nn2 = jnp.sqrt(n2x * n2x + n2y * n2y + n2z * n2z) + _EPS
    ctau = jnp.clip(dotn / (nn1 * nn2), -1.0 + _EPS, 1.0 - _EPS)
    tau = jnp.arctan2(jnp.sqrt(jnp.maximum(1.0 - ctau * ctau, 0.0)), ctau)

    def gf(w_ref, b_ref):
        w = w_ref[...]
        return jnp.maximum(d * w[0:1, :] + theta * w[1:2, :] + phi * w[2:3, :]
                           + tau * w[3:4, :] + b_ref[...], 0.0)

    g1 = gf(w1_ref, b1_ref)
    g2 = gf(w2_ref, b2_ref)
    o1a[...] = g1[:, :64]
    o1b[...] = g1[:, 64:]
    o2a[...] = g2[:, :64]
    o2b[...] = g2[:, 64:]


def _trig_gf(vd, w1, b1, w2, b2):
    be = 4096
    grid = (_EPAD // be,)
    half = jax.ShapeDtypeStruct((_EPAD, 64), jnp.float32)
    return pl.pallas_call(
        _trig_gf_body,
        grid=grid,
        in_specs=[
            pl.BlockSpec((be, 16), lambda i: (i, 0)),
            pl.BlockSpec((4, _H), lambda i: (0, 0)),
            pl.BlockSpec((1, _H), lambda i: (0, 0)),
            pl.BlockSpec((4, _H), lambda i: (0, 0)),
            pl.BlockSpec((1, _H), lambda i: (0, 0)),
        ],
        out_specs=[pl.BlockSpec((be, 64), lambda i: (i, 0))] * 4,
        out_shape=[half, half, half, half],
    )(vd, w1, b1, w2, b2)


def _nodelin_body(h_ref, w_ref, b_ref, oa, ob):
    hh = jnp.dot(h_ref[...], w_ref[...],
                 preferred_element_type=jnp.float32) + b_ref[...]
    oa[...] = hh[:, :64]
    ob[...] = hh[:, 64:]


def _nodelin(h, w, b):
    bn = 2000
    half = jax.ShapeDtypeStruct((_N, 64), jnp.float32)
    return pl.pallas_call(
        _nodelin_body,
        grid=(_N // bn,),
        in_specs=[
            pl.BlockSpec((bn, _H), lambda i: (i, 0)),
            pl.BlockSpec((_H, _H), lambda i: (0, 0)),
            pl.BlockSpec((1, _H), lambda i: (0, 0)),
        ],
        out_specs=[pl.BlockSpec((bn, 64), lambda i: (i, 0))] * 2,
        out_shape=[half, half],
    )(h, w, b)


def _combine_body(s1a, s1b, aa, ab, deg_ref, g2wa, g2wb, g2b, lw, lb, oa, ob):
    acont = (jnp.dot(aa[...], g2wa[...], preferred_element_type=jnp.float32)
             + jnp.dot(ab[...], g2wb[...], preferred_element_type=jnp.float32))
    s1 = jnp.concatenate([s1a[...], s1b[...]], axis=1)
    h = jnp.maximum(s1 + acont + deg_ref[...] * g2b[...], 0.0)
    hh = jnp.dot(h, lw[...], preferred_element_type=jnp.float32) + lb[...]
    oa[...] = hh[:, :64]
    ob[...] = hh[:, 64:]


def _combine_lin(s1a, s1b, aa, ab, deg, g2w, g2b, lw, lb):
    bn = 2000
    half = jax.ShapeDtypeStruct((_N, 64), jnp.float32)
    return pl.pallas_call(
        _combine_body,
        grid=(_N // bn,),
        in_specs=[
            pl.BlockSpec((bn, 64), lambda i: (i, 0)),
            pl.BlockSpec((bn, 64), lambda i: (i, 0)),
            pl.BlockSpec((bn, 64), lambda i: (i, 0)),
            pl.BlockSpec((bn, 64), lambda i: (i, 0)),
            pl.BlockSpec((bn, 1), lambda i: (i, 0)),
            pl.BlockSpec((64, _H), lambda i: (0, 0)),
            pl.BlockSpec((64, _H), lambda i: (0, 0)),
            pl.BlockSpec((1, _H), lambda i: (0, 0)),
            pl.BlockSpec((_H, _H), lambda i: (0, 0)),
            pl.BlockSpec((1, _H), lambda i: (0, 0)),
        ],
        out_specs=[pl.BlockSpec((bn, 64), lambda i: (i, 0))] * 2,
        out_shape=[half, half],
    )(s1a, s1b, aa, ab, deg, g2w[:64], g2w[64:], g2b, lw, lb)


def _final_body(s1a, s1b, aa, ab, deg_ref, g2wa, g2wb, g2b, saw, sab,
                oh_ref, l1w, l1b, l2w, l2b, sums_ref, cnt_ref, out_ref):
    i = pl.program_id(0)
    nsteps = pl.num_programs(0)
    acont = (jnp.dot(aa[...], g2wa[...], preferred_element_type=jnp.float32)
             + jnp.dot(ab[...], g2wb[...], preferred_element_type=jnp.float32))
    s1 = jnp.concatenate([s1a[...], s1b[...]], axis=1)
    h2 = jnp.maximum(s1 + acont + deg_ref[...] * g2b[...], 0.0)
    h3 = jnp.maximum(jnp.dot(h2, saw[...],
                             preferred_element_type=jnp.float32) + sab[...], 0.0)
    oh = oh_ref[...]
    psum = jnp.dot(oh, h3, preferred_element_type=jnp.float32)
    ones = jnp.ones((oh.shape[1], _H), jnp.float32)
    pcnt = jnp.dot(oh, ones, preferred_element_type=jnp.float32)

    @pl.when(i == 0)
    def _():
        sums_ref[...] = psum
        cnt_ref[...] = pcnt

    @pl.when(i > 0)
    def _():
        sums_ref[...] += psum
        cnt_ref[...] += pcnt

    @pl.when(i == nsteps - 1)
    def _():
        cnt = cnt_ref[...]
        sums = sums_ref[...]
        pooled = jnp.where(cnt > 0, sums / jnp.maximum(cnt, 1.0), 0.0)
        hf = jnp.maximum(jnp.dot(pooled, l1w[...],
                                 preferred_element_type=jnp.float32) + l1b[...], 0.0)
        out_ref[...] = jnp.dot(hf, l2w[...],
                               preferred_element_type=jnp.float32) + l2b[...]


def _final_head(s1a, s1b, aa, ab, deg, g2w, g2b, saw, sab, onehot,
                l1w, l1b, l2wp, l2bp):
    bn = 2048
    npad = 10240
    pad = lambda t: jnp.pad(t, ((0, npad - _N), (0, 0)))
    s1a, s1b, aa, ab, deg = map(pad, (s1a, s1b, aa, ab, deg))
    onehot = jnp.pad(onehot, ((0, 0), (0, npad - _N)))
    outs = [jax.ShapeDtypeStruct((_NG, _H), jnp.float32),
            jax.ShapeDtypeStruct((_NG, _H), jnp.float32),
            jax.ShapeDtypeStruct((_NG, _H), jnp.float32)]
    sums, cnt, out = pl.pallas_call(
        _final_body,
        grid=(npad // bn,),
        in_specs=[
            pl.BlockSpec((bn, 64), lambda i: (i, 0)),
            pl.BlockSpec((bn, 64), lambda i: (i, 0)),
            pl.BlockSpec((bn, 64), lambda i: (i, 0)),
            pl.BlockSpec((bn, 64), lambda i: (i, 0)),
            pl.BlockSpec((bn, 1), lambda i: (i, 0)),
            pl.BlockSpec((64, _H), lambda i: (0, 0)),
            pl.BlockSpec((64, _H), lambda i: (0, 0)),
            pl.BlockSpec((1, _H), lambda i: (0, 0)),
            pl.BlockSpec((_H, _H), lambda i: (0, 0)),
            pl.BlockSpec((1, _H), lambda i: (0, 0)),
            pl.BlockSpec((_NG, bn), lambda i: (0, i)),
            pl.BlockSpec((_H, 64), lambda i: (0, 0)),
            pl.BlockSpec((1, 64), lambda i: (0, 0)),
            pl.BlockSpec((64, _H), lambda i: (0, 0)),
            pl.BlockSpec((1, _H), lambda i: (0, 0)),
        ],
        out_specs=[pl.BlockSpec((_NG, _H), lambda i: (0, 0))] * 3,
        out_shape=outs,
    )(s1a, s1b, aa, ab, deg, g2w[:64], g2w[64:], g2b, saw, sab, onehot,
      l1w, l1b, l2wp, l2bp)
    return out[:, :1]


_NPAD = 10240
_CH = 80
_NW = 32
_EPW = _E // _NW
_NCHUNK = _EPW // _CH


_SEG = 2000
_NSEG = _EPW // _SEG
_GRP = _SEG // 16
_SCPARAMS = pltpu.CompilerParams(use_tc_tiling_on_sc=False,
                                 needs_layout_passes=False)


def _rmw_min(tbl_v, idx16, val16, act16):
    t0 = plsc.load_gather(tbl_v, [idx16])
    m0 = act16 & (val16 < t0)
    cnt0 = jnp.sum(m0.astype(jnp.int32))

    def cond(cnt):
        return cnt > 0

    def body(cnt):
        t = plsc.load_gather(tbl_v, [idx16])
        m = act16 & (val16 < t)
        plsc.store_scatter(tbl_v, [idx16], val16, mask=m)
        t2 = plsc.load_gather(tbl_v, [idx16])
        m2 = act16 & (val16 < t2)
        return jnp.sum(m2.astype(jnp.int32))

    lax.while_loop(cond, body, cnt0)


def _tile_combine(tbl_v, sh_tbl, rbuf, racc, out_h, c, s, kind):
    zsl = _NPAD // 16
    pltpu.sync_copy(tbl_v, sh_tbl.at[s])
    plsc.subcore_barrier()
    pltpu.sync_copy(sh_tbl.at[0, pl.ds(s * zsl, zsl)], racc)
    for r in range(1, 16):
        pltpu.sync_copy(sh_tbl.at[r, pl.ds(s * zsl, zsl)], rbuf)

        def red(q, _, r=r):
            a = racc[pl.ds(q * 16, 16)]
            b = rbuf[pl.ds(q * 16, 16)]
            racc[pl.ds(q * 16, 16)] = (jnp.minimum(a, b) if kind == "min"
                                       else a + b)
            return 0

        lax.fori_loop(0, zsl // 16, red, 0)
    pltpu.sync_copy(racc, out_h.at[pl.ds(c * _NPAD + s * zsl, zsl)])


def _sc_geo1(posx, posy, posz, srcf, dstf, bigf):
    mesh = plsc.VectorSubcoreMesh(core_axis_name="c", subcore_axis_name="s")
    out_type = [jax.ShapeDtypeStruct((_E,), jnp.float32),
                jax.ShapeDtypeStruct((2 * _NPAD,), jnp.float32),
                jax.ShapeDtypeStruct((2 * _NPAD,), jnp.float32)]
    scratch = [pltpu.VMEM((_NPAD,), jnp.float32)] * 3 \
        + [pltpu.VMEM((_SEG,), jnp.int32)] * 2 \
        + [pltpu.VMEM((_SEG,), jnp.float32)] \
        + [pltpu.VMEM((_NPAD,), jnp.float32)] * 2 \
        + [pltpu.VMEM_SHARED((16, _NPAD), jnp.float32)] \
        + [pltpu.VMEM((_NPAD // 16,), jnp.float32)] * 2

    @functools.partial(pl.kernel, mesh=mesh, out_type=out_type,
                       scratch_types=scratch, compiler_params=_SCPARAMS)
    def k(posx_h, posy_h, posz_h, src_h, dst_h, big_h, d2_o, min1_o, deg_o,
          px, py, pz, sb, db, d2b, tmin, tdeg, sh_tbl, rbuf, racc):
        c = lax.axis_index("c")
        s = lax.axis_index("s")
        w = s * 2 + c
        pltpu.sync_copy(posx_h, px)
        pltpu.sync_copy(posy_h, py)
        pltpu.sync_copy(posz_h, pz)
        pltpu.sync_copy(big_h, tmin)
        ones16 = jnp.full((16,), 1.0, jnp.float32)

        def zero(q, _):
            tdeg[pl.ds(q * 16, 16)] = jnp.zeros((16,), jnp.float32)
            return 0

        lax.fori_loop(0, _NPAD // 16, zero, 0)
        true16 = jnp.full((16,), True)
        for seg in range(_NSEG):
            base = w * _EPW + seg * _SEG
            pltpu.sync_copy(src_h.at[pl.ds(base, _SEG)], sb)
            pltpu.sync_copy(dst_h.at[pl.ds(base, _SEG)], db)

            def grp(i, _):
                s16 = sb[pl.ds(i * 16, 16)]
                t16 = db[pl.ds(i * 16, 16)]
                dx = (plsc.load_gather(px, [t16])
                      - plsc.load_gather(px, [s16]))
                dy = (plsc.load_gather(py, [t16])
                      - plsc.load_gather(py, [s16]))
                dz = (plsc.load_gather(pz, [t16])
                      - plsc.load_gather(pz, [s16]))
                d2 = dx * dx + dy * dy + dz * dz
                d2b[pl.ds(i * 16, 16)] = d2
                _rmw_min(tmin, s16, d2, true16)
                plsc.addupdate_scatter(tdeg, [t16], ones16)
                return 0

            lax.fori_loop(0, _GRP, grp, 0, unroll=2)
            pltpu.sync_copy(d2b, d2_o.at[pl.ds(base, _SEG)])
        _tile_combine(tmin, sh_tbl, rbuf, racc, min1_o, c, s, "min")
        plsc.subcore_barrier()
        _tile_combine(tdeg, sh_tbl, rbuf, racc, deg_o, c, s, "sum")

    return k(posx, posy, posz, srcf, dstf, bigf)


def _sc_rmw_pass(mode, d2e, srcf, dstf, init_tbl, glbs):
    tdt = jnp.int32 if mode in ("nn1", "nn2") else jnp.float32
    gdts = {"nn1": [jnp.float32], "min2": [jnp.int32],
            "nn2": [jnp.int32, jnp.float32]}[mode]
    nglb = len(glbs)
    mesh = plsc.VectorSubcoreMesh(core_axis_name="c", subcore_axis_name="s")
    out_type = jax.ShapeDtypeStruct((2 * _NPAD,), tdt)
    scratch = ([pltpu.VMEM((_SEG,), jnp.int32)] * 2
               + [pltpu.VMEM((_SEG,), jnp.float32)]
               + [pltpu.VMEM((_NPAD,), g) for g in gdts]
               + [pltpu.VMEM((_NPAD,), tdt)]
               + [pltpu.VMEM_SHARED((16, _NPAD), tdt)]
               + [pltpu.VMEM((_NPAD // 16,), tdt)] * 2)

    @functools.partial(pl.kernel, mesh=mesh, out_type=out_type,
                       scratch_types=scratch, compiler_params=_SCPARAMS)
    def k(*refs):
        d2_h, src_h, dst_h, init_h = refs[:4]
        glb_h = refs[4:4 + nglb]
        out_o = refs[4 + nglb]
        sb, db, d2b = refs[5 + nglb:8 + nglb]
        glb_v = refs[8 + nglb:8 + 2 * nglb]
        tloc = refs[8 + 2 * nglb]
        sh_tbl, rbuf, racc = refs[9 + 2 * nglb:12 + 2 * nglb]
        c = lax.axis_index("c")
        s = lax.axis_index("s")
        w = s * 2 + c
        for gh, gv in zip(glb_h, glb_v):
            pltpu.sync_copy(gh, gv)
        pltpu.sync_copy(init_h, tloc)
        for seg in range(_NSEG):
            base = w * _EPW + seg * _SEG
            pltpu.sync_copy(src_h.at[pl.ds(base, _SEG)], sb)
            pltpu.sync_copy(dst_h.at[pl.ds(base, _SEG)], db)
            pltpu.sync_copy(d2_h.at[pl.ds(base, _SEG)], d2b)

            def grp(i, _):
                s16 = sb[pl.ds(i * 16, 16)]
                t16 = db[pl.ds(i * 16, 16)]
                d216 = d2b[pl.ds(i * 16, 16)]
                if mode == "nn1":
                    act = d216 == plsc.load_gather(glb_v[0], [s16])
                    val = t16
                elif mode == "min2":
                    act = plsc.load_gather(glb_v[0], [s16]) != t16
                    val = d216
                else:
                    act = ((plsc.load_gather(glb_v[0], [s16]) != t16)
                           & (d216 == plsc.load_gather(glb_v[1], [s16])))
                    val = t16
                _rmw_min(tloc, s16, val, act)
                return 0

            lax.fori_loop(0, _GRP, grp, 0, unroll=2)
        _tile_combine(tloc, sh_tbl, rbuf, racc, out_o, c, s, "min")

    return k(d2e, srcf, dstf, init_tbl, *glbs)


def _sc_geo5(posx, posy, posz, srcf, dstf, nn1, nn2, min1, min2):
    mesh = plsc.VectorSubcoreMesh(core_axis_name="c", subcore_axis_name="s")
    out_type = jax.ShapeDtypeStruct((_EPAD, 16), jnp.float32)
    scratch = ([pltpu.VMEM((_NPAD,), jnp.float32)] * 3
               + [pltpu.VMEM((_NPAD,), jnp.int32)] * 2
               + [pltpu.VMEM((_NPAD,), jnp.float32)] * 2
               + [pltpu.VMEM((_SEG,), jnp.int32)] * 2
               + [pltpu.VMEM((_SEG, 16), jnp.float32)])

    @functools.partial(pl.kernel, mesh=mesh, out_type=out_type,
                       scratch_types=scratch, compiler_params=_SCPARAMS)
    def k(posx_h, posy_h, posz_h, src_h, dst_h, nn1_h, nn2_h, m1_h, m2_h,
          vd_o, px, py, pz, n1v, n2v, m1v, m2v, sb, db, stag):
        c = lax.axis_index("c")
        s = lax.axis_index("s")
        w = s * 2 + c
        pltpu.sync_copy(posx_h, px)
        pltpu.sync_copy(posy_h, py)
        pltpu.sync_copy(posz_h, pz)
        pltpu.sync_copy(nn1_h, n1v)
        pltpu.sync_copy(nn2_h, n2v)
        pltpu.sync_copy(m1_h, m1v)
        pltpu.sync_copy(m2_h, m2v)
        lanes = lax.iota(jnp.int32, 16)
        for seg in range(_NSEG):
            base = w * _EPW + seg * _SEG
            pltpu.sync_copy(src_h.at[pl.ds(base, _SEG)], sb)
            pltpu.sync_copy(dst_h.at[pl.ds(base, _SEG)], db)

            def grp(i, _):
                s16 = sb[pl.ds(i * 16, 16)]
                t16 = db[pl.ds(i * 16, 16)]
                lg = plsc.load_gather
                n1s, n2s = lg(n1v, [s16]), lg(n2v, [s16])
                h1s, h2s = lg(m1v, [s16]) < _BIG, lg(m2v, [s16]) < _BIG
                n1d, n2d = lg(n1v, [t16]), lg(n2v, [t16])
                h1d, h2d = lg(m1v, [t16]) < _BIG, lg(m2v, [t16]) < _BIG
                fi = jnp.where(~h1s, s16,
                               jnp.where(n1s != t16, n1s,
                                         jnp.where(h2s, n2s, s16)))
                fj = jnp.where(~h1d, t16,
                               jnp.where(n1d != s16, n1d,
                                         jnp.where(h2d, n2d, t16)))
                pxs, pys, pzs = lg(px, [s16]), lg(py, [s16]), lg(pz, [s16])
                pxt, pyt, pzt = lg(px, [t16]), lg(py, [t16]), lg(pz, [t16])
                v1 = (pxs - lg(px, [fi]), pys - lg(py, [fi]),
                      pzs - lg(pz, [fi]))
                v2 = (pxt - pxs, pyt - pys, pzt - pzs)
                v3 = (pxt - lg(px, [fj]), pyt - lg(py, [fj]),
                      pzt - lg(pz, [fj]))
                d2 = v2[0] * v2[0] + v2[1] * v2[1] + v2[2] * v2[2]
                rows16 = lanes + i * 16
                cols = v1 + v2 + v3 + (d2,)
                for colid, cv in enumerate(cols):
                    plsc.store_scatter(
                        stag, [rows16, jnp.full((16,), colid, jnp.int32)], cv)
                return 0

            lax.fori_loop(0, _GRP, grp, 0, unroll=2)
            pltpu.sync_copy(stag, vd_o.at[pl.ds(base, _SEG)])

    return k(posx, posy, posz, srcf, dstf, nn1, nn2, min1, min2)


def _sc_agg_call(tbl, gfs, src3, dst3, zeros):
    ngf = len(gfs)
    ntbl = 0 if tbl is None else 1
    nacc = ntbl + ngf
    assert nacc <= 2
    R = 5 if nacc == 1 else 1
    mesh = plsc.VectorSubcoreMesh(core_axis_name="c", subcore_axis_name="s")
    out_type = [jax.ShapeDtypeStruct((2 * _NPAD, 64), jnp.float32)] * nacc
    scratch = ([pltpu.VMEM((_NCHUNK, _CH), jnp.int32),
                pltpu.VMEM((_NCHUNK, _CH), jnp.int32)]
               + [pltpu.VMEM((R, _CH, 64), jnp.float32)] * nacc
               + [pltpu.VMEM_SHARED((_NPAD, 64), jnp.float32)] * nacc
               + [pltpu.SemaphoreType.DMA] * (R * nacc))

    @functools.partial(pl.kernel, mesh=mesh, out_type=out_type,
                       scratch_types=scratch, compiler_params=_SCPARAMS)
    def k(*refs):
        tbl_h = refs[0] if ntbl else None
        rest = refs[ntbl:]
        gf_h = rest[:ngf]
        s3, d3, zer_h = rest[ngf:ngf + 3]
        outs = rest[ngf + 3:ngf + 3 + nacc]
        sidx, didx = rest[ngf + 3 + nacc:ngf + 5 + nacc]
        rings = rest[ngf + 5 + nacc:ngf + 5 + 2 * nacc]
        accs = rest[ngf + 5 + 2 * nacc:ngf + 5 + 3 * nacc]
        sems = rest[ngf + 5 + 3 * nacc:]
        c = lax.axis_index("c")
        s = lax.axis_index("s")
        w = s * 2 + c
        zslice = _NPAD // 16
        for acc in accs:
            pltpu.sync_copy(zer_h.at[pl.ds(s * zslice, zslice)],
                            acc.at[pl.ds(s * zslice, zslice)])
        pltpu.sync_copy(s3.at[w], sidx)
        pltpu.sync_copy(d3.at[w], didx)
        plsc.subcore_barrier()

        def load_desc(a, r, j):
            if a < ntbl:
                src = tbl_h.at[sidx.at[j]]
            else:
                src = gf_h[a - ntbl].at[w * _NCHUNK + j]
            return pltpu.make_async_copy(src, rings[a].at[r],
                                         sems[a * R + r])

        for r in range(R):
            for a in range(nacc):
                load_desc(a, r, r).start()

        def round_(j0, carry):
            for r in range(R):
                j = j0 * R + r
                for a in range(nacc):
                    load_desc(a, r, j).wait()
                di = didx.at[j]
                for a in range(nacc):
                    pltpu.sync_copy(rings[a].at[r], accs[a].at[di], add=True)
                jn = j + R

                @pl.when(jn < _NCHUNK)
                def _(r=r, jn=jn):
                    for a in range(nacc):
                        load_desc(a, r, jn).start()

            return carry

        lax.fori_loop(0, _NCHUNK // R, round_, 0)
        plsc.subcore_barrier()
        for acc, out in zip(accs, outs):
            pltpu.sync_copy(acc.at[pl.ds(s * zslice, zslice)],
                            out.at[pl.ds(c * _NPAD + s * zslice, zslice)])

    args = ([] if tbl is None else [tbl]) + list(gfs) + [src3, dst3, zeros]
    res = k(*args)
    res = res if isinstance(res, (list, tuple)) else [res]
    return [r[:_NPAD] + r[_NPAD:] for r in res]


def kernel(x, edge_index, batch, pos, c1_lin_w, c1_lin_b, c1_g1_w, c1_g1_b,
           c1_g2_w, c1_g2_b, c2_lin_w, c2_lin_b, c2_g1_w, c2_g1_b, c2_g2_w,
           c2_g2_b, sa_w, sa_b, l1_w, l1_b, l2_w, l2_b):
    src = edge_index[0].astype(jnp.int32)
    dst = edge_index[1].astype(jnp.int32)

    posp = jnp.pad(pos, ((0, _NPAD - _N), (0, 0)))
    posx, posy, posz = posp[:, 0], posp[:, 1], posp[:, 2]
    bigf = jnp.full((_NPAD,), _BIG, jnp.float32)
    sentn = jnp.full((_NPAD,), _N, jnp.int32)
    d2e, min1p, degp = _sc_geo1(posx, posy, posz, src, dst, bigf)
    min1 = jnp.minimum(min1p[:_NPAD], min1p[_NPAD:])
    deg = (degp[:_NPAD] + degp[_NPAD:])[:_N, None]
    nn1p = _sc_rmw_pass("nn1", d2e, src, dst, sentn, [min1])
    nn1 = jnp.minimum(nn1p[:_NPAD], nn1p[_NPAD:])
    min2p = _sc_rmw_pass("min2", d2e, src, dst, bigf, [nn1])
    min2 = jnp.minimum(min2p[:_NPAD], min2p[_NPAD:])
    nn2p = _sc_rmw_pass("nn2", d2e, src, dst, sentn, [nn1, min2])
    nn2 = jnp.minimum(nn2p[:_NPAD], nn2p[_NPAD:])
    vd = _sc_geo5(posx, posy, posz, src, dst, nn1, nn2, min1, min2)

    gf1a, gf1b, gf2a, gf2b = _trig_gf(
        vd, c1_g1_w, c1_g1_b.reshape(1, _H), c2_g1_w, c2_g1_b.reshape(1, _H))

    src3 = src.reshape(_NW, _NCHUNK, _CH)
    dst3 = dst.reshape(_NW, _NCHUNK, _CH)
    zeros = jnp.zeros((_NPAD, 64), jnp.float32)
    rs = lambda g: g[:_E].reshape(_E // _CH, _CH, 64)

    hh1a, hh1b = _nodelin(x, c1_lin_w, c1_lin_b.reshape(1, _H))
    (s1a,) = _sc_agg_call(hh1a, [], src3, dst3, zeros)
    (s1b,) = _sc_agg_call(hh1b, [], src3, dst3, zeros)
    (a1a,) = _sc_agg_call(None, [rs(gf1a)], src3, dst3, zeros)
    (a2a,) = _sc_agg_call(None, [rs(gf2a)], src3, dst3, zeros)
    (a1b,) = _sc_agg_call(None, [rs(gf1b)], src3, dst3, zeros)
    (a2b,) = _sc_agg_call(None, [rs(gf2b)], src3, dst3, zeros)
    s1a, s1b, a1a, a1b = s1a[:_N], s1b[:_N], a1a[:_N], a1b[:_N]

    hh2a, hh2b = _combine_lin(s1a, s1b, a1a, a1b, deg, c1_g2_w,
                              c1_g2_b.reshape(1, _H), c2_lin_w,
                              c2_lin_b.reshape(1, _H))
    (s2a,) = _sc_agg_call(hh2a, [], src3, dst3, zeros)
    (s2b,) = _sc_agg_call(hh2b, [], src3, dst3, zeros)
    s2a, s2b, a2a, a2b = s2a[:_N], s2b[:_N], a2a[:_N], a2b[:_N]

    onehot = (batch[None, :].astype(jnp.int32)
              == jnp.arange(_NG, dtype=jnp.int32)[:, None]).astype(jnp.float32)
    l2wp = jnp.pad(l2_w, ((0, 0), (0, _H - 1)))
    l2bp = jnp.pad(l2_b, (0, _H - 1)).reshape(1, _H)
    return _final_head(s2a, s2b, a2a, a2b, deg, c2_g2_w,
                       c2_g2_b.reshape(1, _H), sa_w, sa_b.reshape(1, _H),
                       onehot, l1_w, l1_b.reshape(1, 64), l2wp, l2bp)
